# Initial kernel scaffold; baseline (speedup 1.0000x reference)
import jax
import jax.numpy as jnp
from jax.experimental import pallas as pl


def kernel(x, edge_index, batch, params):
    raise NotImplementedError("write your pallas kernel here")


# speedup vs baseline: 9.8633x; 9.8633x over previous
import functools

import jax
import jax.numpy as jnp
from jax import lax
from jax.experimental import pallas as pl
from jax.experimental.pallas import tpu as pltpu
from jax.experimental.pallas import tpu_sc as plsc

_N = 10000
_E = 320000
_DIN = 128
_H = 64
_DOUT = 128
_NG = 16

_NC = 2
_NS = 16
_NW = _NC * _NS
_NPAD = 10240
_RPT = _NPAD // _NS
_CHUNK = 128
_EW = 10240
_NCHUNK = _EW // _CHUNK
_EPAD = _NW * _EW

_CW = 16
_R = 1024
_GRID = _NPAD // _R


def _dotT(a, w):
    return lax.dot_general(a, w, (((1,), (1,)), ((), ())),
                           preferred_element_type=jnp.float32)


def _sc_agg_body(with_deg, *refs):
    if with_deg:
        (tbl_hbm, src_hbm, dst_hbm, zeros_hbm, ones_hbm, zeros1_hbm,
         s_hbm, cnt_hbm,
         src_v, dst_v, buf, ones_v, acc_s, cnt_s, gsem) = refs
    else:
        (tbl_hbm, src_hbm, dst_hbm, zeros_hbm,
         s_hbm,
         src_v, dst_v, buf, acc_s, gsem) = refs
    cid = lax.axis_index("c")
    sid = lax.axis_index("s")
    w = cid * _NS + sid
    rbase = sid * _RPT
    rows = pl.ds(rbase, _RPT)

    pltpu.sync_copy(zeros_hbm, acc_s.at[rows])
    pltpu.sync_copy(src_hbm.at[w], src_v)
    pltpu.sync_copy(dst_hbm.at[w], dst_v)
    if with_deg:
        pltpu.sync_copy(zeros1_hbm, cnt_s.at[rows])
        pltpu.sync_copy(ones_hbm, ones_v)
    plsc.subcore_barrier()

    def body(j, carry):
        pltpu.async_copy(tbl_hbm.at[src_v.at[j]], buf, gsem).wait()
        pltpu.sync_copy(buf, acc_s.at[dst_v.at[j]], add=True)
        if with_deg:
            pltpu.sync_copy(ones_v, cnt_s.at[dst_v.at[j]], add=True)
        return carry

    lax.fori_loop(0, _NCHUNK, body, 0)
    plsc.subcore_barrier()

    pltpu.sync_copy(acc_s.at[rows], s_hbm.at[cid, rows])
    if with_deg:
        pltpu.sync_copy(cnt_s.at[rows], cnt_hbm.at[cid, rows])


@functools.cache
def _make_sc_agg(with_deg):
    scratch = [
        pltpu.VMEM((_NCHUNK, _CHUNK), jnp.int32),
        pltpu.VMEM((_NCHUNK, _CHUNK), jnp.int32),
        pltpu.VMEM((_CHUNK, _H), jnp.float32),
    ]
    out_type = [jax.ShapeDtypeStruct((_NC, _NPAD, _H), jnp.float32)]
    if with_deg:
        scratch.append(pltpu.VMEM((_CHUNK, _CW), jnp.float32))
        out_type.append(jax.ShapeDtypeStruct((_NC, _NPAD, _CW), jnp.float32))
    scratch += [
        pltpu.VMEM_SHARED((_NPAD, _H), jnp.float32),
    ]
    if with_deg:
        scratch.append(pltpu.VMEM_SHARED((_NPAD, _CW), jnp.float32))
    scratch.append(pltpu.SemaphoreType.DMA)
    mesh = plsc.VectorSubcoreMesh(core_axis_name="c", subcore_axis_name="s")
    return pl.kernel(
        functools.partial(_sc_agg_body, with_deg),
        out_type=tuple(out_type),
        mesh=mesh,
        scratch_types=scratch,
        compiler_params=pltpu.CompilerParams(use_tc_tiling_on_sc=False),
        name="sc_agg_deg" if with_deg else "sc_agg",
    )


def _sc_agg_deg(*args):
    return _make_sc_agg(True)(*args)


def _sc_agg(*args):
    return _make_sc_agg(False)(*args)


def _tc_pre_body(x_ref, wl_ref, wr_ref, bl_ref, y_ref, r_ref):
    x = x_ref[...]
    y_ref[...] = _dotT(x, wl_ref[...])
    r_ref[...] = _dotT(x, wr_ref[...]) + bl_ref[...]


def _tc_pre(x_pad, wl, wr, bl):
    return pl.pallas_call(
        _tc_pre_body,
        grid=(_GRID,),
        in_specs=[
            pl.BlockSpec((_R, _DIN), lambda i: (i, 0)),
            pl.BlockSpec((_H, _DIN), lambda i: (0, 0)),
            pl.BlockSpec((_H, _DIN), lambda i: (0, 0)),
            pl.BlockSpec((1, _H), lambda i: (0, 0)),
        ],
        out_specs=[
            pl.BlockSpec((_R, _H), lambda i: (i, 0)),
            pl.BlockSpec((_R, _H), lambda i: (i, 0)),
        ],
        out_shape=[
            jax.ShapeDtypeStruct((_NPAD, _H), jnp.float32),
            jax.ShapeDtypeStruct((_NPAD, _H), jnp.float32),
        ],
    )(x_pad, wl, wr, bl)


def _layer_post(s, cnt, r, g, b):
    inv = 1.0 / jnp.maximum(cnt[0] + cnt[1], 1.0)
    t = (s[0] + s[1]) * inv + r
    mu = jnp.mean(t, axis=-1, keepdims=True)
    d = t - mu
    var = jnp.mean(d * d, axis=-1, keepdims=True)
    hn = d * lax.rsqrt(var + 1e-5) * g + b
    return jnp.maximum(hn, 0.0)


def _tc_mid_body(s_ref, cnt_ref, r_ref, g_ref, b_ref, wl_ref, bl_ref,
                 wr_ref, y_ref, r2_ref):
    h = _layer_post(s_ref[...], cnt_ref[...], r_ref[...], g_ref[...],
                    b_ref[...])
    y_ref[...] = _dotT(h, wl_ref[...])
    r2_ref[...] = _dotT(h, wr_ref[...]) + bl_ref[...]


def _tc_mid(s, cnt, r, g, b, wl, bl, wr):
    return pl.pallas_call(
        _tc_mid_body,
        grid=(_GRID,),
        in_specs=[
            pl.BlockSpec((_NC, _R, _H), lambda i: (0, i, 0)),
            pl.BlockSpec((_NC, _R, 1), lambda i: (0, i, 0)),
            pl.BlockSpec((_R, _H), lambda i: (i, 0)),
            pl.BlockSpec((1, _H), lambda i: (0, 0)),
            pl.BlockSpec((1, _H), lambda i: (0, 0)),
            pl.BlockSpec((_H, _H), lambda i: (0, 0)),
            pl.BlockSpec((1, _H), lambda i: (0, 0)),
            pl.BlockSpec((_H, _H), lambda i: (0, 0)),
        ],
        out_specs=[
            pl.BlockSpec((_R, _H), lambda i: (i, 0)),
            pl.BlockSpec((_R, _H), lambda i: (i, 0)),
        ],
        out_shape=[
            jax.ShapeDtypeStruct((_NPAD, _H), jnp.float32),
            jax.ShapeDtypeStruct((_NPAD, _H), jnp.float32),
        ],
    )(s, cnt, r, g, b, wl, bl, wr)


def _tc_last_body(s_ref, cnt_ref, r_ref, g_ref, b_ref, wrf_ref, blf_ref,
                  y_ref, r2_ref):
    h = _layer_post(s_ref[...], cnt_ref[...], r_ref[...], g_ref[...],
                    b_ref[...])
    y_ref[...] = h
    r2_ref[...] = _dotT(h, wrf_ref[...]) + blf_ref[...]


def _tc_last(s, cnt, r, g, b, wrf, blf):
    return pl.pallas_call(
        _tc_last_body,
        grid=(_GRID,),
        in_specs=[
            pl.BlockSpec((_NC, _R, _H), lambda i: (0, i, 0)),
            pl.BlockSpec((_NC, _R, 1), lambda i: (0, i, 0)),
            pl.BlockSpec((_R, _H), lambda i: (i, 0)),
            pl.BlockSpec((1, _H), lambda i: (0, 0)),
            pl.BlockSpec((1, _H), lambda i: (0, 0)),
            pl.BlockSpec((_DOUT, _H), lambda i: (0, 0)),
            pl.BlockSpec((1, _DOUT), lambda i: (0, 0)),
        ],
        out_specs=[
            pl.BlockSpec((_R, _H), lambda i: (i, 0)),
            pl.BlockSpec((_R, _DOUT), lambda i: (i, 0)),
        ],
        out_shape=[
            jax.ShapeDtypeStruct((_NPAD, _H), jnp.float32),
            jax.ShapeDtypeStruct((_NPAD, _DOUT), jnp.float32),
        ],
    )(s, cnt, r, g, b, wrf, blf)


def _tc_final_body(s_ref, cnt_ref, r_ref, wlf_ref, batch_ref, wo_ref, bo_ref,
                   out_ref, acc, accc):
    step = pl.program_id(0)
    s = s_ref[...]
    inv = 1.0 / jnp.maximum(cnt_ref[0] + cnt_ref[1], 1.0)
    mean = (s[0] + s[1]) * inv
    hf = _dotT(mean, wlf_ref[...]) + r_ref[...]
    bt = batch_ref[...]
    oh = (bt == lax.broadcasted_iota(jnp.int32, (_R, _NG), 1))
    oh = oh.astype(jnp.float32)
    part = lax.dot_general(oh, hf, (((0,), (0,)), ((), ())),
                           preferred_element_type=jnp.float32)
    pc = lax.dot_general(oh, jnp.ones((_R, 1), jnp.float32),
                         (((0,), (0,)), ((), ())),
                         preferred_element_type=jnp.float32)

    @pl.when(step == 0)
    def _():
        acc[...] = jnp.zeros_like(acc)
        accc[...] = jnp.zeros_like(accc)

    acc[...] += part
    accc[...] += pc

    @pl.when(step == pl.num_programs(0) - 1)
    def _():
        gm = acc[...] / jnp.maximum(accc[...], 1.0)
        out_ref[...] = _dotT(gm, wo_ref[...]) + bo_ref[...]


def _tc_final(s, cnt, r, wlf, batch2d, wo, bo):
    return pl.pallas_call(
        _tc_final_body,
        grid=(_GRID,),
        in_specs=[
            pl.BlockSpec((_NC, _R, _H), lambda i: (0, i, 0)),
            pl.BlockSpec((_NC, _R, 1), lambda i: (0, i, 0)),
            pl.BlockSpec((_R, _DOUT), lambda i: (i, 0)),
            pl.BlockSpec((_DOUT, _H), lambda i: (0, 0)),
            pl.BlockSpec((_R, 1), lambda i: (i, 0)),
            pl.BlockSpec((_DOUT, _DOUT), lambda i: (0, 0)),
            pl.BlockSpec((1, _DOUT), lambda i: (0, 0)),
        ],
        out_specs=pl.BlockSpec((_NG, _DOUT), lambda i: (0, 0)),
        out_shape=jax.ShapeDtypeStruct((_NG, _DOUT), jnp.float32),
        scratch_shapes=[
            pltpu.VMEM((_NG, _DOUT), jnp.float32),
            pltpu.VMEM((_NG, 1), jnp.float32),
        ],
    )(s, cnt, r, wlf, batch2d, wo, bo)


def kernel(x, edge_index, batch, params):
    f32 = jnp.float32
    x_pad = jnp.zeros((_NPAD, _DIN), f32).at[:_N].set(x)

    npad_rows = _NPAD - _N
    pad_idx = _N + (jnp.arange(_EPAD - _E, dtype=jnp.int32) % npad_rows)
    src = jnp.concatenate([edge_index[0], pad_idx]).reshape(_NW, _NCHUNK, _CHUNK)
    dst = jnp.concatenate([edge_index[1], pad_idx]).reshape(_NW, _NCHUNK, _CHUNK)

    zeros64 = jnp.zeros((_RPT, _H), f32)
    zeros1 = jnp.zeros((_RPT, _CW), f32)
    ones1 = jnp.ones((_CHUNK, _CW), f32)
    batch2d = jnp.full((_NPAD, 1), _NG, jnp.int32).at[:_N, 0].set(batch)

    def w2(name):
        return params[name]

    def b2(name):
        return params[name].reshape(1, -1)

    y, r = _tc_pre(x_pad, w2("Wl0"), w2("Wr0"), b2("bl0"))
    s, cnt = _sc_agg_deg(y, src, dst, zeros64, ones1, zeros1)
    cnt = cnt[:, :, :1]
    for i in range(1, 5):
        y, r = _tc_mid(s, cnt, r, b2(f"g{i-1}"), b2(f"b{i-1}"),
                       w2(f"Wl{i}"), b2(f"bl{i}"), w2(f"Wr{i}"))
        (s,) = _sc_agg(y, src, dst, zeros64)
    y, r = _tc_last(s, cnt, r, b2("g4"), b2("b4"), w2("Wrf"), b2("blf"))
    (s,) = _sc_agg(y, src, dst, zeros64)
    out = _tc_final(s, cnt, r, w2("Wlf"), batch2d, w2("Wo"), b2("bo"))
    return out


# speedup vs baseline: 15.3367x; 1.5549x over previous
import functools

import jax
import jax.numpy as jnp
from jax import lax
from jax.experimental import pallas as pl
from jax.experimental.pallas import tpu as pltpu
from jax.experimental.pallas import tpu_sc as plsc

_N = 10000
_E = 320000
_DIN = 128
_H = 64
_DOUT = 128
_NG = 16

_NC = 2
_NS = 16
_NW = _NC * _NS
_NPAD = 10240
_RPT = _NPAD // _NS
_CHUNK = 128
_EW = 10240
_NCHUNK = _EW // _CHUNK
_EPAD = _NW * _EW

_CW = 16
_R = 1024
_GRID = _NPAD // _R


def _dotT(a, w):
    return lax.dot_general(a, w, (((1,), (1,)), ((), ())),
                           preferred_element_type=jnp.float32)


_NBUF = 4


def _sc_agg_body(with_deg, *refs):
    if with_deg:
        (tbl_hbm, src_hbm, dst_hbm, zeros_hbm, ones_hbm, zeros1_hbm,
         s_hbm, cnt_hbm,
         src_v, dst_v, *rest) = refs
        bufs = rest[:_NBUF]
        ones_v, acc_s, cnt_s = rest[_NBUF:_NBUF + 3]
        gsems = rest[_NBUF + 3:2 * _NBUF + 3]
        ssems = rest[2 * _NBUF + 3:3 * _NBUF + 3]
        osems = rest[3 * _NBUF + 3:]
    else:
        (tbl_hbm, src_hbm, dst_hbm, zeros_hbm,
         s_hbm,
         src_v, dst_v, *rest) = refs
        bufs = rest[:_NBUF]
        acc_s = rest[_NBUF]
        gsems = rest[_NBUF + 1:2 * _NBUF + 1]
        ssems = rest[2 * _NBUF + 1:]
    cid = lax.axis_index("c")
    sid = lax.axis_index("s")
    w = cid * _NS + sid
    rbase = sid * _RPT
    rows = pl.ds(rbase, _RPT)

    def g_start(b, c):
        pltpu.async_copy(tbl_hbm.at[src_v.at[c]], bufs[b], gsems[b])

    def g_wait(b):
        pltpu.make_async_copy(tbl_hbm.at[src_v.at[0]], bufs[b],
                              gsems[b]).wait()

    def s_start(b, c):
        pltpu.async_copy(bufs[b], acc_s.at[dst_v.at[c]], ssems[b], add=True)
        if with_deg:
            pltpu.async_copy(ones_v, cnt_s.at[dst_v.at[c]], osems[b],
                             add=True)

    def s_wait(b):
        pltpu.make_async_copy(bufs[b], acc_s.at[dst_v.at[0]], ssems[b]).wait()
        if with_deg:
            pltpu.make_async_copy(ones_v, cnt_s.at[dst_v.at[0]],
                                  osems[b]).wait()

    pltpu.sync_copy(zeros_hbm, acc_s.at[rows])
    pltpu.sync_copy(src_hbm.at[w], src_v)
    pltpu.sync_copy(dst_hbm.at[w], dst_v)
    if with_deg:
        pltpu.sync_copy(zeros1_hbm, cnt_s.at[rows])
        pltpu.sync_copy(ones_hbm, ones_v)
    plsc.subcore_barrier()

    for b in range(_NBUF):
        g_start(b, b)

    def round_body(r, carry):
        j = r * _NBUF
        for b in range(_NBUF):
            g_wait(b)
            s_start(b, j + b)
        for b in range(_NBUF):
            nxt = j + b + _NBUF

            @pl.when(nxt < _NCHUNK)
            def _(b=b, nxt=nxt):
                s_wait(b)
                g_start(b, nxt)
        return carry

    lax.fori_loop(0, _NCHUNK // _NBUF, round_body, 0)
    for b in range(_NBUF):
        s_wait(b)
    plsc.subcore_barrier()

    pltpu.sync_copy(acc_s.at[rows], s_hbm.at[cid, rows])
    if with_deg:
        pltpu.sync_copy(cnt_s.at[rows], cnt_hbm.at[cid, rows])


@functools.cache
def _make_sc_agg(with_deg):
    scratch = [
        pltpu.VMEM((_NCHUNK, _CHUNK), jnp.int32),
        pltpu.VMEM((_NCHUNK, _CHUNK), jnp.int32),
    ]
    scratch += [pltpu.VMEM((_CHUNK, _H), jnp.float32)] * _NBUF
    out_type = [jax.ShapeDtypeStruct((_NC, _NPAD, _H), jnp.float32)]
    if with_deg:
        scratch.append(pltpu.VMEM((_CHUNK, _CW), jnp.float32))
        out_type.append(jax.ShapeDtypeStruct((_NC, _NPAD, _CW), jnp.float32))
    scratch += [
        pltpu.VMEM_SHARED((_NPAD, _H), jnp.float32),
    ]
    if with_deg:
        scratch.append(pltpu.VMEM_SHARED((_NPAD, _CW), jnp.float32))
    scratch += [pltpu.SemaphoreType.DMA] * (2 * _NBUF)
    if with_deg:
        scratch += [pltpu.SemaphoreType.DMA] * _NBUF
    mesh = plsc.VectorSubcoreMesh(core_axis_name="c", subcore_axis_name="s")
    return pl.kernel(
        functools.partial(_sc_agg_body, with_deg),
        out_type=tuple(out_type),
        mesh=mesh,
        scratch_types=scratch,
        compiler_params=pltpu.CompilerParams(use_tc_tiling_on_sc=False),
        name="sc_agg_deg" if with_deg else "sc_agg",
    )


def _sc_agg_deg(*args):
    return _make_sc_agg(True)(*args)


def _sc_agg(*args):
    return _make_sc_agg(False)(*args)


def _tc_pre_body(x_ref, wl_ref, wr_ref, bl_ref, y_ref, r_ref):
    x = x_ref[...]
    y_ref[...] = _dotT(x, wl_ref[...])
    r_ref[...] = _dotT(x, wr_ref[...]) + bl_ref[...]


def _tc_pre(x_pad, wl, wr, bl):
    return pl.pallas_call(
        _tc_pre_body,
        grid=(_GRID,),
        in_specs=[
            pl.BlockSpec((_R, _DIN), lambda i: (i, 0)),
            pl.BlockSpec((_H, _DIN), lambda i: (0, 0)),
            pl.BlockSpec((_H, _DIN), lambda i: (0, 0)),
            pl.BlockSpec((1, _H), lambda i: (0, 0)),
        ],
        out_specs=[
            pl.BlockSpec((_R, _H), lambda i: (i, 0)),
            pl.BlockSpec((_R, _H), lambda i: (i, 0)),
        ],
        out_shape=[
            jax.ShapeDtypeStruct((_NPAD, _H), jnp.float32),
            jax.ShapeDtypeStruct((_NPAD, _H), jnp.float32),
        ],
    )(x_pad, wl, wr, bl)


def _layer_post(s, cnt, r, g, b):
    inv = 1.0 / jnp.maximum(cnt[0] + cnt[1], 1.0)
    t = (s[0] + s[1]) * inv + r
    mu = jnp.mean(t, axis=-1, keepdims=True)
    d = t - mu
    var = jnp.mean(d * d, axis=-1, keepdims=True)
    hn = d * lax.rsqrt(var + 1e-5) * g + b
    return jnp.maximum(hn, 0.0)


def _tc_mid_body(s_ref, cnt_ref, r_ref, g_ref, b_ref, wl_ref, bl_ref,
                 wr_ref, y_ref, r2_ref):
    h = _layer_post(s_ref[...], cnt_ref[...], r_ref[...], g_ref[...],
                    b_ref[...])
    y_ref[...] = _dotT(h, wl_ref[...])
    r2_ref[...] = _dotT(h, wr_ref[...]) + bl_ref[...]


def _tc_mid(s, cnt, r, g, b, wl, bl, wr):
    return pl.pallas_call(
        _tc_mid_body,
        grid=(_GRID,),
        in_specs=[
            pl.BlockSpec((_NC, _R, _H), lambda i: (0, i, 0)),
            pl.BlockSpec((_NC, _R, 1), lambda i: (0, i, 0)),
            pl.BlockSpec((_R, _H), lambda i: (i, 0)),
            pl.BlockSpec((1, _H), lambda i: (0, 0)),
            pl.BlockSpec((1, _H), lambda i: (0, 0)),
            pl.BlockSpec((_H, _H), lambda i: (0, 0)),
            pl.BlockSpec((1, _H), lambda i: (0, 0)),
            pl.BlockSpec((_H, _H), lambda i: (0, 0)),
        ],
        out_specs=[
            pl.BlockSpec((_R, _H), lambda i: (i, 0)),
            pl.BlockSpec((_R, _H), lambda i: (i, 0)),
        ],
        out_shape=[
            jax.ShapeDtypeStruct((_NPAD, _H), jnp.float32),
            jax.ShapeDtypeStruct((_NPAD, _H), jnp.float32),
        ],
    )(s, cnt, r, g, b, wl, bl, wr)


def _tc_last_body(s_ref, cnt_ref, r_ref, g_ref, b_ref, wrf_ref, blf_ref,
                  y_ref, r2_ref):
    h = _layer_post(s_ref[...], cnt_ref[...], r_ref[...], g_ref[...],
                    b_ref[...])
    y_ref[...] = h
    r2_ref[...] = _dotT(h, wrf_ref[...]) + blf_ref[...]


def _tc_last(s, cnt, r, g, b, wrf, blf):
    return pl.pallas_call(
        _tc_last_body,
        grid=(_GRID,),
        in_specs=[
            pl.BlockSpec((_NC, _R, _H), lambda i: (0, i, 0)),
            pl.BlockSpec((_NC, _R, 1), lambda i: (0, i, 0)),
            pl.BlockSpec((_R, _H), lambda i: (i, 0)),
            pl.BlockSpec((1, _H), lambda i: (0, 0)),
            pl.BlockSpec((1, _H), lambda i: (0, 0)),
            pl.BlockSpec((_DOUT, _H), lambda i: (0, 0)),
            pl.BlockSpec((1, _DOUT), lambda i: (0, 0)),
        ],
        out_specs=[
            pl.BlockSpec((_R, _H), lambda i: (i, 0)),
            pl.BlockSpec((_R, _DOUT), lambda i: (i, 0)),
        ],
        out_shape=[
            jax.ShapeDtypeStruct((_NPAD, _H), jnp.float32),
            jax.ShapeDtypeStruct((_NPAD, _DOUT), jnp.float32),
        ],
    )(s, cnt, r, g, b, wrf, blf)


def _tc_final_body(s_ref, cnt_ref, r_ref, wlf_ref, batch_ref, wo_ref, bo_ref,
                   out_ref, acc, accc):
    step = pl.program_id(0)
    s = s_ref[...]
    inv = 1.0 / jnp.maximum(cnt_ref[0] + cnt_ref[1], 1.0)
    mean = (s[0] + s[1]) * inv
    hf = _dotT(mean, wlf_ref[...]) + r_ref[...]
    bt = batch_ref[...]
    oh = (bt == lax.broadcasted_iota(jnp.int32, (_R, _NG), 1))
    oh = oh.astype(jnp.float32)
    part = lax.dot_general(oh, hf, (((0,), (0,)), ((), ())),
                           preferred_element_type=jnp.float32)
    pc = lax.dot_general(oh, jnp.ones((_R, 1), jnp.float32),
                         (((0,), (0,)), ((), ())),
                         preferred_element_type=jnp.float32)

    @pl.when(step == 0)
    def _():
        acc[...] = jnp.zeros_like(acc)
        accc[...] = jnp.zeros_like(accc)

    acc[...] += part
    accc[...] += pc

    @pl.when(step == pl.num_programs(0) - 1)
    def _():
        gm = acc[...] / jnp.maximum(accc[...], 1.0)
        out_ref[...] = _dotT(gm, wo_ref[...]) + bo_ref[...]


def _tc_final(s, cnt, r, wlf, batch2d, wo, bo):
    return pl.pallas_call(
        _tc_final_body,
        grid=(_GRID,),
        in_specs=[
            pl.BlockSpec((_NC, _R, _H), lambda i: (0, i, 0)),
            pl.BlockSpec((_NC, _R, 1), lambda i: (0, i, 0)),
            pl.BlockSpec((_R, _DOUT), lambda i: (i, 0)),
            pl.BlockSpec((_DOUT, _H), lambda i: (0, 0)),
            pl.BlockSpec((_R, 1), lambda i: (i, 0)),
            pl.BlockSpec((_DOUT, _DOUT), lambda i: (0, 0)),
            pl.BlockSpec((1, _DOUT), lambda i: (0, 0)),
        ],
        out_specs=pl.BlockSpec((_NG, _DOUT), lambda i: (0, 0)),
        out_shape=jax.ShapeDtypeStruct((_NG, _DOUT), jnp.float32),
        scratch_shapes=[
            pltpu.VMEM((_NG, _DOUT), jnp.float32),
            pltpu.VMEM((_NG, 1), jnp.float32),
        ],
    )(s, cnt, r, wlf, batch2d, wo, bo)


def kernel(x, edge_index, batch, params):
    f32 = jnp.float32
    x_pad = jnp.zeros((_NPAD, _DIN), f32).at[:_N].set(x)

    npad_rows = _NPAD - _N
    pad_idx = _N + (jnp.arange(_EPAD - _E, dtype=jnp.int32) % npad_rows)
    src = jnp.concatenate([edge_index[0], pad_idx]).reshape(_NW, _NCHUNK, _CHUNK)
    dst = jnp.concatenate([edge_index[1], pad_idx]).reshape(_NW, _NCHUNK, _CHUNK)

    zeros64 = jnp.zeros((_RPT, _H), f32)
    zeros1 = jnp.zeros((_RPT, _CW), f32)
    ones1 = jnp.ones((_CHUNK, _CW), f32)
    batch2d = jnp.full((_NPAD, 1), _NG, jnp.int32).at[:_N, 0].set(batch)

    def w2(name):
        return params[name]

    def b2(name):
        return params[name].reshape(1, -1)

    y, r = _tc_pre(x_pad, w2("Wl0"), w2("Wr0"), b2("bl0"))
    s, cnt = _sc_agg_deg(y, src, dst, zeros64, ones1, zeros1)
    cnt = cnt[:, :, :1]
    for i in range(1, 5):
        y, r = _tc_mid(s, cnt, r, b2(f"g{i-1}"), b2(f"b{i-1}"),
                       w2(f"Wl{i}"), b2(f"bl{i}"), w2(f"Wr{i}"))
        (s,) = _sc_agg(y, src, dst, zeros64)
    y, r = _tc_last(s, cnt, r, b2("g4"), b2("b4"), w2("Wrf"), b2("blf"))
    (s,) = _sc_agg(y, src, dst, zeros64)
    out = _tc_final(s, cnt, r, w2("Wlf"), batch2d, w2("Wo"), b2("bo"))
    return out


# speedup vs baseline: 15.6288x; 1.0191x over previous
import functools

import jax
import jax.numpy as jnp
from jax import lax
from jax.experimental import pallas as pl
from jax.experimental.pallas import tpu as pltpu
from jax.experimental.pallas import tpu_sc as plsc

_N = 10000
_E = 320000
_DIN = 128
_H = 64
_DOUT = 128
_NG = 16

_NC = 2
_NS = 16
_NW = _NC * _NS
_NPAD = 10240
_RPT = _NPAD // _NS
_CHUNK = 128
_EW = 10240
_NCHUNK = _EW // _CHUNK
_EPAD = _NW * _EW

_CW = 16
_R = 1024
_GRID = _NPAD // _R


def _dotT(a, w):
    return lax.dot_general(a, w, (((1,), (1,)), ((), ())),
                           preferred_element_type=jnp.float32)


_NBUF = 4


def _sc_agg_body(with_deg, *refs):
    if with_deg:
        (tbl_hbm, src_hbm, dst_hbm, zeros_hbm, ones_hbm, zeros1_hbm,
         s_hbm, cnt_hbm,
         src_v, dst_v, *rest) = refs
        bufs = rest[:_NBUF]
        ones_v, acc_s, cnt_s = rest[_NBUF:_NBUF + 3]
        gsems = rest[_NBUF + 3:2 * _NBUF + 3]
        ssems = rest[2 * _NBUF + 3:3 * _NBUF + 3]
        osems = rest[3 * _NBUF + 3:]
    else:
        (tbl_hbm, src_hbm, dst_hbm, zeros_hbm,
         s_hbm,
         src_v, dst_v, *rest) = refs
        bufs = rest[:_NBUF]
        acc_s = rest[_NBUF]
        gsems = rest[_NBUF + 1:2 * _NBUF + 1]
        ssems = rest[2 * _NBUF + 1:]
    cid = lax.axis_index("c")
    sid = lax.axis_index("s")
    w = cid * _NS + sid
    rbase = sid * _RPT
    rows = pl.ds(rbase, _RPT)

    def g_start(b, c):
        pltpu.async_copy(tbl_hbm.at[src_v.at[c]], bufs[b], gsems[b])

    def g_wait(b):
        pltpu.make_async_copy(tbl_hbm.at[src_v.at[0]], bufs[b],
                              gsems[b]).wait()

    def s_start(b, c):
        pltpu.async_copy(bufs[b], acc_s.at[dst_v.at[c]], ssems[b], add=True)
        if with_deg:
            pltpu.async_copy(ones_v, cnt_s.at[dst_v.at[c]], osems[b],
                             add=True)

    def s_wait(b):
        pltpu.make_async_copy(bufs[b], acc_s.at[dst_v.at[0]], ssems[b]).wait()
        if with_deg:
            pltpu.make_async_copy(ones_v, cnt_s.at[dst_v.at[0]],
                                  osems[b]).wait()

    pltpu.sync_copy(zeros_hbm, acc_s.at[rows])
    pltpu.sync_copy(src_hbm.at[w], src_v)
    pltpu.sync_copy(dst_hbm.at[w], dst_v)
    if with_deg:
        pltpu.sync_copy(zeros1_hbm, cnt_s.at[rows])
        pltpu.sync_copy(ones_hbm, ones_v)
    plsc.subcore_barrier()

    for b in range(_NBUF):
        g_start(b, b)

    def round_body(r, carry):
        j = r * _NBUF
        for b in range(_NBUF):
            g_wait(b)
            s_start(b, j + b)
        for b in range(_NBUF):
            nxt = j + b + _NBUF

            @pl.when(nxt < _NCHUNK)
            def _(b=b, nxt=nxt):
                s_wait(b)
                g_start(b, nxt)
        return carry

    lax.fori_loop(0, _NCHUNK // _NBUF, round_body, 0)
    for b in range(_NBUF):
        s_wait(b)
    plsc.subcore_barrier()

    pltpu.sync_copy(acc_s.at[rows], s_hbm.at[cid, rows])
    if with_deg:
        pltpu.sync_copy(cnt_s.at[rows], cnt_hbm.at[cid, rows])


_NBUF2 = 8
_NGRP = _NCHUNK // _NBUF2


def _sc_agg_body2(tbl_hbm, src_hbm, dst_hbm, zeros_hbm, s_hbm,
                  src_v, dst_v, *rest):
    bufs = rest[:_NBUF2]
    acc_s = rest[_NBUF2]
    gsems = rest[_NBUF2 + 1:2 * _NBUF2 + 1]
    ssems = rest[2 * _NBUF2 + 1:3 * _NBUF2 + 1]
    isrc = rest[3 * _NBUF2 + 1:3 * _NBUF2 + 3]
    idst = rest[3 * _NBUF2 + 3:]
    cid = lax.axis_index("c")
    sid = lax.axis_index("s")
    w = cid * _NS + sid
    rows = pl.ds(sid * _RPT, _RPT)

    def stage(grp, slot):
        pltpu.async_copy(src_hbm.at[w, pl.ds(grp * _NBUF2, _NBUF2)],
                         src_v.at[slot], isrc[slot])
        pltpu.async_copy(dst_hbm.at[w, pl.ds(grp * _NBUF2, _NBUF2)],
                         dst_v.at[slot], idst[slot])

    def stage_wait(slot):
        pltpu.make_async_copy(src_hbm.at[w, pl.ds(0, _NBUF2)],
                              src_v.at[slot], isrc[slot]).wait()
        pltpu.make_async_copy(dst_hbm.at[w, pl.ds(0, _NBUF2)],
                              dst_v.at[slot], idst[slot]).wait()

    def g_start(b, slot):
        pltpu.async_copy(tbl_hbm.at[src_v.at[slot, b]], bufs[b], gsems[b])

    def g_wait(b):
        pltpu.make_async_copy(tbl_hbm.at[src_v.at[0, 0]], bufs[b],
                              gsems[b]).wait()

    def s_start(b, slot):
        pltpu.async_copy(bufs[b], acc_s.at[dst_v.at[slot, b]], ssems[b],
                         add=True)

    def s_wait(b):
        pltpu.make_async_copy(bufs[b], acc_s.at[dst_v.at[0, 0]],
                              ssems[b]).wait()

    pltpu.sync_copy(zeros_hbm, acc_s.at[rows])
    stage(0, 0)
    stage(1, 1)
    plsc.subcore_barrier()
    stage_wait(0)
    for b in range(_NBUF2):
        g_start(b, 0)

    def do_round(r, slot, nslot):
        for b in range(_NBUF2):
            g_wait(b)
            s_start(b, slot)

        @pl.when(r + 1 < _NGRP)
        def _():
            stage_wait(nslot)
            for b in range(_NBUF2):
                s_wait(b)
                g_start(b, nslot)

        @pl.when(r + 2 < _NGRP)
        def _():
            stage(r + 2, slot)

    def pair_body(p, carry):
        do_round(2 * p, 0, 1)
        do_round(2 * p + 1, 1, 0)
        return carry

    lax.fori_loop(0, _NGRP // 2, pair_body, 0)
    for b in range(_NBUF2):
        s_wait(b)
    plsc.subcore_barrier()
    pltpu.sync_copy(acc_s.at[rows], s_hbm.at[cid, rows])


@functools.cache
def _make_sc_agg(with_deg):
    if not with_deg:
        scratch = [
            pltpu.VMEM((2, _NBUF2, _CHUNK), jnp.int32),
            pltpu.VMEM((2, _NBUF2, _CHUNK), jnp.int32),
        ]
        scratch += [pltpu.VMEM((_CHUNK, _H), jnp.float32)] * _NBUF2
        scratch.append(pltpu.VMEM_SHARED((_NPAD, _H), jnp.float32))
        scratch += [pltpu.SemaphoreType.DMA] * (2 * _NBUF2 + 4)
        mesh = plsc.VectorSubcoreMesh(core_axis_name="c",
                                      subcore_axis_name="s")
        return pl.kernel(
            _sc_agg_body2,
            out_type=(jax.ShapeDtypeStruct((_NC, _NPAD, _H), jnp.float32),),
            mesh=mesh,
            scratch_types=scratch,
            compiler_params=pltpu.CompilerParams(use_tc_tiling_on_sc=False),
            name="sc_agg",
        )
    scratch = [
        pltpu.VMEM((_NCHUNK, _CHUNK), jnp.int32),
        pltpu.VMEM((_NCHUNK, _CHUNK), jnp.int32),
    ]
    scratch += [pltpu.VMEM((_CHUNK, _H), jnp.float32)] * _NBUF
    out_type = [jax.ShapeDtypeStruct((_NC, _NPAD, _H), jnp.float32)]
    if with_deg:
        scratch.append(pltpu.VMEM((_CHUNK, _CW), jnp.float32))
        out_type.append(jax.ShapeDtypeStruct((_NC, _NPAD, _CW), jnp.float32))
    scratch += [
        pltpu.VMEM_SHARED((_NPAD, _H), jnp.float32),
    ]
    if with_deg:
        scratch.append(pltpu.VMEM_SHARED((_NPAD, _CW), jnp.float32))
    scratch += [pltpu.SemaphoreType.DMA] * (2 * _NBUF)
    if with_deg:
        scratch += [pltpu.SemaphoreType.DMA] * _NBUF
    mesh = plsc.VectorSubcoreMesh(core_axis_name="c", subcore_axis_name="s")
    return pl.kernel(
        functools.partial(_sc_agg_body, with_deg),
        out_type=tuple(out_type),
        mesh=mesh,
        scratch_types=scratch,
        compiler_params=pltpu.CompilerParams(use_tc_tiling_on_sc=False),
        name="sc_agg_deg" if with_deg else "sc_agg",
    )


def _sc_agg_deg(*args):
    return _make_sc_agg(True)(*args)


def _sc_agg(*args):
    return _make_sc_agg(False)(*args)


def _tc_pre_body(x_ref, wl_ref, wr_ref, bl_ref, y_ref, r_ref):
    x = x_ref[...]
    y_ref[...] = _dotT(x, wl_ref[...])
    r_ref[...] = _dotT(x, wr_ref[...]) + bl_ref[...]


def _tc_pre(x_pad, wl, wr, bl):
    return pl.pallas_call(
        _tc_pre_body,
        grid=(_GRID,),
        in_specs=[
            pl.BlockSpec((_R, _DIN), lambda i: (i, 0)),
            pl.BlockSpec((_H, _DIN), lambda i: (0, 0)),
            pl.BlockSpec((_H, _DIN), lambda i: (0, 0)),
            pl.BlockSpec((1, _H), lambda i: (0, 0)),
        ],
        out_specs=[
            pl.BlockSpec((_R, _H), lambda i: (i, 0)),
            pl.BlockSpec((_R, _H), lambda i: (i, 0)),
        ],
        out_shape=[
            jax.ShapeDtypeStruct((_NPAD, _H), jnp.float32),
            jax.ShapeDtypeStruct((_NPAD, _H), jnp.float32),
        ],
    )(x_pad, wl, wr, bl)


def _layer_post(s, cnt, r, g, b):
    inv = 1.0 / jnp.maximum(cnt[0] + cnt[1], 1.0)
    t = (s[0] + s[1]) * inv + r
    mu = jnp.mean(t, axis=-1, keepdims=True)
    d = t - mu
    var = jnp.mean(d * d, axis=-1, keepdims=True)
    hn = d * lax.rsqrt(var + 1e-5) * g + b
    return jnp.maximum(hn, 0.0)


def _tc_mid_body(s_ref, cnt_ref, r_ref, g_ref, b_ref, wl_ref, bl_ref,
                 wr_ref, y_ref, r2_ref):
    h = _layer_post(s_ref[...], cnt_ref[...], r_ref[...], g_ref[...],
                    b_ref[...])
    y_ref[...] = _dotT(h, wl_ref[...])
    r2_ref[...] = _dotT(h, wr_ref[...]) + bl_ref[...]


def _tc_mid(s, cnt, r, g, b, wl, bl, wr):
    return pl.pallas_call(
        _tc_mid_body,
        grid=(_GRID,),
        in_specs=[
            pl.BlockSpec((_NC, _R, _H), lambda i: (0, i, 0)),
            pl.BlockSpec((_NC, _R, 1), lambda i: (0, i, 0)),
            pl.BlockSpec((_R, _H), lambda i: (i, 0)),
            pl.BlockSpec((1, _H), lambda i: (0, 0)),
            pl.BlockSpec((1, _H), lambda i: (0, 0)),
            pl.BlockSpec((_H, _H), lambda i: (0, 0)),
            pl.BlockSpec((1, _H), lambda i: (0, 0)),
            pl.BlockSpec((_H, _H), lambda i: (0, 0)),
        ],
        out_specs=[
            pl.BlockSpec((_R, _H), lambda i: (i, 0)),
            pl.BlockSpec((_R, _H), lambda i: (i, 0)),
        ],
        out_shape=[
            jax.ShapeDtypeStruct((_NPAD, _H), jnp.float32),
            jax.ShapeDtypeStruct((_NPAD, _H), jnp.float32),
        ],
    )(s, cnt, r, g, b, wl, bl, wr)


def _tc_last_body(s_ref, cnt_ref, r_ref, g_ref, b_ref, wrf_ref, blf_ref,
                  y_ref, r2_ref):
    h = _layer_post(s_ref[...], cnt_ref[...], r_ref[...], g_ref[...],
                    b_ref[...])
    y_ref[...] = h
    r2_ref[...] = _dotT(h, wrf_ref[...]) + blf_ref[...]


def _tc_last(s, cnt, r, g, b, wrf, blf):
    return pl.pallas_call(
        _tc_last_body,
        grid=(_GRID,),
        in_specs=[
            pl.BlockSpec((_NC, _R, _H), lambda i: (0, i, 0)),
            pl.BlockSpec((_NC, _R, 1), lambda i: (0, i, 0)),
            pl.BlockSpec((_R, _H), lambda i: (i, 0)),
            pl.BlockSpec((1, _H), lambda i: (0, 0)),
            pl.BlockSpec((1, _H), lambda i: (0, 0)),
            pl.BlockSpec((_DOUT, _H), lambda i: (0, 0)),
            pl.BlockSpec((1, _DOUT), lambda i: (0, 0)),
        ],
        out_specs=[
            pl.BlockSpec((_R, _H), lambda i: (i, 0)),
            pl.BlockSpec((_R, _DOUT), lambda i: (i, 0)),
        ],
        out_shape=[
            jax.ShapeDtypeStruct((_NPAD, _H), jnp.float32),
            jax.ShapeDtypeStruct((_NPAD, _DOUT), jnp.float32),
        ],
    )(s, cnt, r, g, b, wrf, blf)


def _tc_final_body(s_ref, cnt_ref, r_ref, wlf_ref, batch_ref, wo_ref, bo_ref,
                   out_ref, acc, accc):
    step = pl.program_id(0)
    s = s_ref[...]
    inv = 1.0 / jnp.maximum(cnt_ref[0] + cnt_ref[1], 1.0)
    mean = (s[0] + s[1]) * inv
    hf = _dotT(mean, wlf_ref[...]) + r_ref[...]
    bt = batch_ref[...]
    oh = (bt == lax.broadcasted_iota(jnp.int32, (_R, _NG), 1))
    oh = oh.astype(jnp.float32)
    part = lax.dot_general(oh, hf, (((0,), (0,)), ((), ())),
                           preferred_element_type=jnp.float32)
    pc = lax.dot_general(oh, jnp.ones((_R, 1), jnp.float32),
                         (((0,), (0,)), ((), ())),
                         preferred_element_type=jnp.float32)

    @pl.when(step == 0)
    def _():
        acc[...] = jnp.zeros_like(acc)
        accc[...] = jnp.zeros_like(accc)

    acc[...] += part
    accc[...] += pc

    @pl.when(step == pl.num_programs(0) - 1)
    def _():
        gm = acc[...] / jnp.maximum(accc[...], 1.0)
        out_ref[...] = _dotT(gm, wo_ref[...]) + bo_ref[...]


def _tc_final(s, cnt, r, wlf, batch2d, wo, bo):
    return pl.pallas_call(
        _tc_final_body,
        grid=(_GRID,),
        in_specs=[
            pl.BlockSpec((_NC, _R, _H), lambda i: (0, i, 0)),
            pl.BlockSpec((_NC, _R, 1), lambda i: (0, i, 0)),
            pl.BlockSpec((_R, _DOUT), lambda i: (i, 0)),
            pl.BlockSpec((_DOUT, _H), lambda i: (0, 0)),
            pl.BlockSpec((_R, 1), lambda i: (i, 0)),
            pl.BlockSpec((_DOUT, _DOUT), lambda i: (0, 0)),
            pl.BlockSpec((1, _DOUT), lambda i: (0, 0)),
        ],
        out_specs=pl.BlockSpec((_NG, _DOUT), lambda i: (0, 0)),
        out_shape=jax.ShapeDtypeStruct((_NG, _DOUT), jnp.float32),
        scratch_shapes=[
            pltpu.VMEM((_NG, _DOUT), jnp.float32),
            pltpu.VMEM((_NG, 1), jnp.float32),
        ],
    )(s, cnt, r, wlf, batch2d, wo, bo)


def kernel(x, edge_index, batch, params):
    f32 = jnp.float32
    x_pad = jnp.zeros((_NPAD, _DIN), f32).at[:_N].set(x)

    npad_rows = _NPAD - _N
    pad_idx = _N + (jnp.arange(_EPAD - _E, dtype=jnp.int32) % npad_rows)
    src = jnp.concatenate([edge_index[0], pad_idx]).reshape(_NW, _NCHUNK, _CHUNK)
    dst = jnp.concatenate([edge_index[1], pad_idx]).reshape(_NW, _NCHUNK, _CHUNK)

    zeros64 = jnp.zeros((_RPT, _H), f32)
    zeros1 = jnp.zeros((_RPT, _CW), f32)
    ones1 = jnp.ones((_CHUNK, _CW), f32)
    batch2d = jnp.full((_NPAD, 1), _NG, jnp.int32).at[:_N, 0].set(batch)

    def w2(name):
        return params[name]

    def b2(name):
        return params[name].reshape(1, -1)

    y, r = _tc_pre(x_pad, w2("Wl0"), w2("Wr0"), b2("bl0"))
    s, cnt = _sc_agg_deg(y, src, dst, zeros64, ones1, zeros1)
    cnt = cnt[:, :, :1]
    for i in range(1, 5):
        y, r = _tc_mid(s, cnt, r, b2(f"g{i-1}"), b2(f"b{i-1}"),
                       w2(f"Wl{i}"), b2(f"bl{i}"), w2(f"Wr{i}"))
        (s,) = _sc_agg(y, src, dst, zeros64)
    y, r = _tc_last(s, cnt, r, b2("g4"), b2("b4"), w2("Wrf"), b2("blf"))
    (s,) = _sc_agg(y, src, dst, zeros64)
    out = _tc_final(s, cnt, r, w2("Wlf"), batch2d, w2("Wo"), b2("bo"))
    return out


# speedup vs baseline: 19.8563x; 1.2705x over previous
import functools

import jax
import jax.numpy as jnp
from jax import lax
from jax.experimental import pallas as pl
from jax.experimental.pallas import tpu as pltpu
from jax.experimental.pallas import tpu_sc as plsc

_N = 10000
_E = 320000
_DIN = 128
_H = 64
_DOUT = 128
_NG = 16

_NC = 2
_NS = 16
_NW = _NC * _NS
_NPAD = 10240
_RPT = _NPAD // _NS
_CHUNK = 128
_EW = 10240
_NCHUNK = _EW // _CHUNK
_EPAD = _NW * _EW

_CW = 16
_R = 1024
_GRID = _NPAD // _R


def _dotT(a, w):
    return lax.dot_general(a, w, (((1,), (1,)), ((), ())),
                           preferred_element_type=jnp.float32)


_NBUF = 4


def _sc_agg_body(with_deg, *refs):
    if with_deg:
        (tbl_hbm, src_hbm, dst_hbm, zeros_hbm, ones_hbm, zeros1_hbm,
         s_hbm, cnt_hbm,
         src_v, dst_v, *rest) = refs
        bufs = rest[:_NBUF]
        ones_v, acc_s, cnt_s = rest[_NBUF:_NBUF + 3]
        gsems = rest[_NBUF + 3:2 * _NBUF + 3]
        ssems = rest[2 * _NBUF + 3:3 * _NBUF + 3]
        osems = rest[3 * _NBUF + 3:]
    else:
        (tbl_hbm, src_hbm, dst_hbm, zeros_hbm,
         s_hbm,
         src_v, dst_v, *rest) = refs
        bufs = rest[:_NBUF]
        acc_s = rest[_NBUF]
        gsems = rest[_NBUF + 1:2 * _NBUF + 1]
        ssems = rest[2 * _NBUF + 1:]
    cid = lax.axis_index("c")
    sid = lax.axis_index("s")
    w = cid * _NS + sid
    rbase = sid * _RPT
    rows = pl.ds(rbase, _RPT)

    def g_start(b, c):
        pltpu.async_copy(tbl_hbm.at[src_v.at[c]], bufs[b], gsems[b])

    def g_wait(b):
        pltpu.make_async_copy(tbl_hbm.at[src_v.at[0]], bufs[b],
                              gsems[b]).wait()

    def s_start(b, c):
        pltpu.async_copy(bufs[b], acc_s.at[dst_v.at[c]], ssems[b], add=True)
        if with_deg:
            pltpu.async_copy(ones_v, cnt_s.at[dst_v.at[c]], osems[b],
                             add=True)

    def s_wait(b):
        pltpu.make_async_copy(bufs[b], acc_s.at[dst_v.at[0]], ssems[b]).wait()
        if with_deg:
            pltpu.make_async_copy(ones_v, cnt_s.at[dst_v.at[0]],
                                  osems[b]).wait()

    pltpu.sync_copy(zeros_hbm, acc_s.at[rows])
    pltpu.sync_copy(src_hbm.at[w], src_v)
    pltpu.sync_copy(dst_hbm.at[w], dst_v)
    if with_deg:
        pltpu.sync_copy(zeros1_hbm, cnt_s.at[rows])
        pltpu.sync_copy(ones_hbm, ones_v)
    plsc.subcore_barrier()

    for b in range(_NBUF):
        g_start(b, b)

    def round_body(r, carry):
        j = r * _NBUF
        for b in range(_NBUF):
            g_wait(b)
            s_start(b, j + b)
        for b in range(_NBUF):
            nxt = j + b + _NBUF

            @pl.when(nxt < _NCHUNK)
            def _(b=b, nxt=nxt):
                s_wait(b)
                g_start(b, nxt)
        return carry

    lax.fori_loop(0, _NCHUNK // _NBUF, round_body, 0)
    for b in range(_NBUF):
        s_wait(b)
    plsc.subcore_barrier()

    pltpu.sync_copy(acc_s.at[rows], s_hbm.at[cid, rows])
    if with_deg:
        pltpu.sync_copy(cnt_s.at[rows], cnt_hbm.at[cid, rows])


_NBUF2 = 8
_NGRP = _NCHUNK // _NBUF2


def _sc_agg_body2(tbl_hbm, src_hbm, dst_hbm, zeros_hbm, s_hbm,
                  src_v, dst_v, *rest):
    bufs = rest[:_NBUF2]
    acc_s = rest[_NBUF2]
    gsems = rest[_NBUF2 + 1:2 * _NBUF2 + 1]
    ssems = rest[2 * _NBUF2 + 1:3 * _NBUF2 + 1]
    isrc = rest[3 * _NBUF2 + 1:3 * _NBUF2 + 3]
    idst = rest[3 * _NBUF2 + 3:]
    cid = lax.axis_index("c")
    sid = lax.axis_index("s")
    w = cid * _NS + sid
    rows = pl.ds(sid * _RPT, _RPT)

    def stage(grp, slot):
        pltpu.async_copy(src_hbm.at[w, pl.ds(grp * _NBUF2, _NBUF2)],
                         src_v.at[slot], isrc[slot])
        pltpu.async_copy(dst_hbm.at[w, pl.ds(grp * _NBUF2, _NBUF2)],
                         dst_v.at[slot], idst[slot])

    def stage_wait(slot):
        pltpu.make_async_copy(src_hbm.at[w, pl.ds(0, _NBUF2)],
                              src_v.at[slot], isrc[slot]).wait()
        pltpu.make_async_copy(dst_hbm.at[w, pl.ds(0, _NBUF2)],
                              dst_v.at[slot], idst[slot]).wait()

    def g_start(b, slot):
        pltpu.async_copy(tbl_hbm.at[src_v.at[slot, b]], bufs[b], gsems[b])

    def g_wait(b):
        pltpu.make_async_copy(tbl_hbm.at[src_v.at[0, 0]], bufs[b],
                              gsems[b]).wait()

    def s_start(b, slot):
        pltpu.async_copy(bufs[b], acc_s.at[dst_v.at[slot, b]], ssems[b],
                         add=True)

    def s_wait(b):
        pltpu.make_async_copy(bufs[b], acc_s.at[dst_v.at[0, 0]],
                              ssems[b]).wait()

    pltpu.sync_copy(zeros_hbm, acc_s.at[rows])
    stage(0, 0)
    stage(1, 1)
    plsc.subcore_barrier()
    stage_wait(0)
    for b in range(_NBUF2):
        g_start(b, 0)

    def do_round(r, slot, nslot):
        for b in range(_NBUF2):
            g_wait(b)
            s_start(b, slot)

        @pl.when(r + 1 < _NGRP)
        def _():
            stage_wait(nslot)
            for b in range(_NBUF2):
                s_wait(b)
                g_start(b, nslot)

        @pl.when(r + 2 < _NGRP)
        def _():
            stage(r + 2, slot)

    def pair_body(p, carry):
        do_round(2 * p, 0, 1)
        do_round(2 * p + 1, 1, 0)
        return carry

    lax.fori_loop(0, _NGRP // 2, pair_body, 0)
    for b in range(_NBUF2):
        s_wait(b)
    plsc.subcore_barrier()
    pltpu.sync_copy(acc_s.at[rows], s_hbm.at[cid, rows])


@functools.cache
def _make_sc_agg(with_deg):
    if not with_deg:
        scratch = [
            pltpu.VMEM((2, _NBUF2, _CHUNK), jnp.int32),
            pltpu.VMEM((2, _NBUF2, _CHUNK), jnp.int32),
        ]
        scratch += [pltpu.VMEM((_CHUNK, _H), jnp.bfloat16)] * _NBUF2
        scratch.append(pltpu.VMEM_SHARED((_NPAD, _H), jnp.bfloat16))
        scratch += [pltpu.SemaphoreType.DMA] * (2 * _NBUF2 + 4)
        mesh = plsc.VectorSubcoreMesh(core_axis_name="c",
                                      subcore_axis_name="s")
        return pl.kernel(
            _sc_agg_body2,
            out_type=(jax.ShapeDtypeStruct((_NC, _NPAD, _H), jnp.bfloat16),),
            mesh=mesh,
            scratch_types=scratch,
            compiler_params=pltpu.CompilerParams(use_tc_tiling_on_sc=False),
            name="sc_agg",
        )
    scratch = [
        pltpu.VMEM((_NCHUNK, _CHUNK), jnp.int32),
        pltpu.VMEM((_NCHUNK, _CHUNK), jnp.int32),
    ]
    scratch += [pltpu.VMEM((_CHUNK, _H), jnp.bfloat16)] * _NBUF
    out_type = [jax.ShapeDtypeStruct((_NC, _NPAD, _H), jnp.bfloat16)]
    if with_deg:
        scratch.append(pltpu.VMEM((_CHUNK, _CW), jnp.float32))
        out_type.append(jax.ShapeDtypeStruct((_NC, _NPAD, _CW), jnp.float32))
    scratch += [
        pltpu.VMEM_SHARED((_NPAD, _H), jnp.bfloat16),
    ]
    if with_deg:
        scratch.append(pltpu.VMEM_SHARED((_NPAD, _CW), jnp.float32))
    scratch += [pltpu.SemaphoreType.DMA] * (2 * _NBUF)
    if with_deg:
        scratch += [pltpu.SemaphoreType.DMA] * _NBUF
    mesh = plsc.VectorSubcoreMesh(core_axis_name="c", subcore_axis_name="s")
    return pl.kernel(
        functools.partial(_sc_agg_body, with_deg),
        out_type=tuple(out_type),
        mesh=mesh,
        scratch_types=scratch,
        compiler_params=pltpu.CompilerParams(use_tc_tiling_on_sc=False),
        name="sc_agg_deg" if with_deg else "sc_agg",
    )


def _sc_agg_deg(*args):
    return _make_sc_agg(True)(*args)


def _sc_agg(*args):
    return _make_sc_agg(False)(*args)


def _tc_pre_body(x_ref, wl_ref, wr_ref, bl_ref, y_ref, r_ref):
    x = x_ref[...]
    y_ref[...] = _dotT(x, wl_ref[...]).astype(jnp.bfloat16)
    r_ref[...] = _dotT(x, wr_ref[...]) + bl_ref[...]


def _tc_pre(x_pad, wl, wr, bl):
    return pl.pallas_call(
        _tc_pre_body,
        grid=(_GRID,),
        in_specs=[
            pl.BlockSpec((_R, _DIN), lambda i: (i, 0)),
            pl.BlockSpec((_H, _DIN), lambda i: (0, 0)),
            pl.BlockSpec((_H, _DIN), lambda i: (0, 0)),
            pl.BlockSpec((1, _H), lambda i: (0, 0)),
        ],
        out_specs=[
            pl.BlockSpec((_R, _H), lambda i: (i, 0)),
            pl.BlockSpec((_R, _H), lambda i: (i, 0)),
        ],
        out_shape=[
            jax.ShapeDtypeStruct((_NPAD, _H), jnp.bfloat16),
            jax.ShapeDtypeStruct((_NPAD, _H), jnp.float32),
        ],
    )(x_pad, wl, wr, bl)


def _layer_post(s, cnt, r, g, b):
    inv = 1.0 / jnp.maximum(cnt[0] + cnt[1], 1.0)
    t = (s[0].astype(jnp.float32) + s[1].astype(jnp.float32)) * inv + r
    mu = jnp.mean(t, axis=-1, keepdims=True)
    d = t - mu
    var = jnp.mean(d * d, axis=-1, keepdims=True)
    hn = d * lax.rsqrt(var + 1e-5) * g + b
    return jnp.maximum(hn, 0.0)


def _tc_mid_body(s_ref, cnt_ref, r_ref, g_ref, b_ref, wl_ref, bl_ref,
                 wr_ref, y_ref, r2_ref):
    h = _layer_post(s_ref[...], cnt_ref[...], r_ref[...], g_ref[...],
                    b_ref[...])
    y_ref[...] = _dotT(h, wl_ref[...]).astype(jnp.bfloat16)
    r2_ref[...] = _dotT(h, wr_ref[...]) + bl_ref[...]


def _tc_mid(s, cnt, r, g, b, wl, bl, wr):
    return pl.pallas_call(
        _tc_mid_body,
        grid=(_GRID,),
        in_specs=[
            pl.BlockSpec((_NC, _R, _H), lambda i: (0, i, 0)),
            pl.BlockSpec((_NC, _R, 1), lambda i: (0, i, 0)),
            pl.BlockSpec((_R, _H), lambda i: (i, 0)),
            pl.BlockSpec((1, _H), lambda i: (0, 0)),
            pl.BlockSpec((1, _H), lambda i: (0, 0)),
            pl.BlockSpec((_H, _H), lambda i: (0, 0)),
            pl.BlockSpec((1, _H), lambda i: (0, 0)),
            pl.BlockSpec((_H, _H), lambda i: (0, 0)),
        ],
        out_specs=[
            pl.BlockSpec((_R, _H), lambda i: (i, 0)),
            pl.BlockSpec((_R, _H), lambda i: (i, 0)),
        ],
        out_shape=[
            jax.ShapeDtypeStruct((_NPAD, _H), jnp.bfloat16),
            jax.ShapeDtypeStruct((_NPAD, _H), jnp.float32),
        ],
    )(s, cnt, r, g, b, wl, bl, wr)


def _tc_last_body(s_ref, cnt_ref, r_ref, g_ref, b_ref, wrf_ref, blf_ref,
                  y_ref, r2_ref):
    h = _layer_post(s_ref[...], cnt_ref[...], r_ref[...], g_ref[...],
                    b_ref[...])
    y_ref[...] = h.astype(jnp.bfloat16)
    r2_ref[...] = _dotT(h, wrf_ref[...]) + blf_ref[...]


def _tc_last(s, cnt, r, g, b, wrf, blf):
    return pl.pallas_call(
        _tc_last_body,
        grid=(_GRID,),
        in_specs=[
            pl.BlockSpec((_NC, _R, _H), lambda i: (0, i, 0)),
            pl.BlockSpec((_NC, _R, 1), lambda i: (0, i, 0)),
            pl.BlockSpec((_R, _H), lambda i: (i, 0)),
            pl.BlockSpec((1, _H), lambda i: (0, 0)),
            pl.BlockSpec((1, _H), lambda i: (0, 0)),
            pl.BlockSpec((_DOUT, _H), lambda i: (0, 0)),
            pl.BlockSpec((1, _DOUT), lambda i: (0, 0)),
        ],
        out_specs=[
            pl.BlockSpec((_R, _H), lambda i: (i, 0)),
            pl.BlockSpec((_R, _DOUT), lambda i: (i, 0)),
        ],
        out_shape=[
            jax.ShapeDtypeStruct((_NPAD, _H), jnp.bfloat16),
            jax.ShapeDtypeStruct((_NPAD, _DOUT), jnp.float32),
        ],
    )(s, cnt, r, g, b, wrf, blf)


def _tc_final_body(s_ref, cnt_ref, r_ref, wlf_ref, batch_ref, wo_ref, bo_ref,
                   out_ref, acc, accc):
    step = pl.program_id(0)
    s = s_ref[...].astype(jnp.float32)
    inv = 1.0 / jnp.maximum(cnt_ref[0] + cnt_ref[1], 1.0)
    mean = (s[0] + s[1]) * inv
    hf = _dotT(mean, wlf_ref[...]) + r_ref[...]
    bt = batch_ref[...]
    oh = (bt == lax.broadcasted_iota(jnp.int32, (_R, _NG), 1))
    oh = oh.astype(jnp.float32)
    part = lax.dot_general(oh, hf, (((0,), (0,)), ((), ())),
                           preferred_element_type=jnp.float32)
    pc = lax.dot_general(oh, jnp.ones((_R, 1), jnp.float32),
                         (((0,), (0,)), ((), ())),
                         preferred_element_type=jnp.float32)

    @pl.when(step == 0)
    def _():
        acc[...] = jnp.zeros_like(acc)
        accc[...] = jnp.zeros_like(accc)

    acc[...] += part
    accc[...] += pc

    @pl.when(step == pl.num_programs(0) - 1)
    def _():
        gm = acc[...] / jnp.maximum(accc[...], 1.0)
        out_ref[...] = _dotT(gm, wo_ref[...]) + bo_ref[...]


def _tc_final(s, cnt, r, wlf, batch2d, wo, bo):
    return pl.pallas_call(
        _tc_final_body,
        grid=(_GRID,),
        in_specs=[
            pl.BlockSpec((_NC, _R, _H), lambda i: (0, i, 0)),
            pl.BlockSpec((_NC, _R, 1), lambda i: (0, i, 0)),
            pl.BlockSpec((_R, _DOUT), lambda i: (i, 0)),
            pl.BlockSpec((_DOUT, _H), lambda i: (0, 0)),
            pl.BlockSpec((_R, 1), lambda i: (i, 0)),
            pl.BlockSpec((_DOUT, _DOUT), lambda i: (0, 0)),
            pl.BlockSpec((1, _DOUT), lambda i: (0, 0)),
        ],
        out_specs=pl.BlockSpec((_NG, _DOUT), lambda i: (0, 0)),
        out_shape=jax.ShapeDtypeStruct((_NG, _DOUT), jnp.float32),
        scratch_shapes=[
            pltpu.VMEM((_NG, _DOUT), jnp.float32),
            pltpu.VMEM((_NG, 1), jnp.float32),
        ],
    )(s, cnt, r, wlf, batch2d, wo, bo)


def kernel(x, edge_index, batch, params):
    f32 = jnp.float32
    x_pad = jnp.zeros((_NPAD, _DIN), f32).at[:_N].set(x)

    npad_rows = _NPAD - _N
    pad_idx = _N + (jnp.arange(_EPAD - _E, dtype=jnp.int32) % npad_rows)
    src = jnp.concatenate([edge_index[0], pad_idx]).reshape(_NW, _NCHUNK, _CHUNK)
    dst = jnp.concatenate([edge_index[1], pad_idx]).reshape(_NW, _NCHUNK, _CHUNK)

    zeros64 = jnp.zeros((_RPT, _H), jnp.bfloat16)
    zeros1 = jnp.zeros((_RPT, _CW), f32)
    ones1 = jnp.ones((_CHUNK, _CW), f32)
    batch2d = jnp.full((_NPAD, 1), _NG, jnp.int32).at[:_N, 0].set(batch)

    def w2(name):
        return params[name]

    def b2(name):
        return params[name].reshape(1, -1)

    y, r = _tc_pre(x_pad, w2("Wl0"), w2("Wr0"), b2("bl0"))
    s, cnt = _sc_agg_deg(y, src, dst, zeros64, ones1, zeros1)
    cnt = cnt[:, :, :1]
    for i in range(1, 5):
        y, r = _tc_mid(s, cnt, r, b2(f"g{i-1}"), b2(f"b{i-1}"),
                       w2(f"Wl{i}"), b2(f"bl{i}"), w2(f"Wr{i}"))
        (s,) = _sc_agg(y, src, dst, zeros64)
    y, r = _tc_last(s, cnt, r, b2("g4"), b2("b4"), w2("Wrf"), b2("blf"))
    (s,) = _sc_agg(y, src, dst, zeros64)
    out = _tc_final(s, cnt, r, w2("Wlf"), batch2d, w2("Wo"), b2("bo"))
    return out


# speedup vs baseline: 20.6119x; 1.0381x over previous
import functools

import jax
import jax.numpy as jnp
from jax import lax
from jax.experimental import pallas as pl
from jax.experimental.pallas import tpu as pltpu
from jax.experimental.pallas import tpu_sc as plsc

_N = 10000
_E = 320000
_DIN = 128
_H = 64
_DOUT = 128
_NG = 16

_NC = 2
_NS = 16
_NW = _NC * _NS
_NPAD = 10240
_RPT = _NPAD // _NS
_CHUNK = 128
_EW = 10240
_NCHUNK = _EW // _CHUNK
_EPAD = _NW * _EW

_CW = 16
_R = 2048
_GRID = _NPAD // _R


def _dotT(a, w):
    return lax.dot_general(a, w, (((1,), (1,)), ((), ())),
                           preferred_element_type=jnp.float32)


_NBUF = 4


def _sc_agg_body(with_deg, *refs):
    if with_deg:
        (tbl_hbm, src_hbm, dst_hbm, zeros_hbm, ones_hbm, zeros1_hbm,
         s_hbm, cnt_hbm,
         src_v, dst_v, *rest) = refs
        bufs = rest[:_NBUF]
        ones_v, acc_s, cnt_s = rest[_NBUF:_NBUF + 3]
        gsems = rest[_NBUF + 3:2 * _NBUF + 3]
        ssems = rest[2 * _NBUF + 3:3 * _NBUF + 3]
        osems = rest[3 * _NBUF + 3:]
    else:
        (tbl_hbm, src_hbm, dst_hbm, zeros_hbm,
         s_hbm,
         src_v, dst_v, *rest) = refs
        bufs = rest[:_NBUF]
        acc_s = rest[_NBUF]
        gsems = rest[_NBUF + 1:2 * _NBUF + 1]
        ssems = rest[2 * _NBUF + 1:]
    cid = lax.axis_index("c")
    sid = lax.axis_index("s")
    w = cid * _NS + sid
    rbase = sid * _RPT
    rows = pl.ds(rbase, _RPT)

    def g_start(b, c):
        pltpu.async_copy(tbl_hbm.at[src_v.at[c]], bufs[b], gsems[b])

    def g_wait(b):
        pltpu.make_async_copy(tbl_hbm.at[src_v.at[0]], bufs[b],
                              gsems[b]).wait()

    def s_start(b, c):
        pltpu.async_copy(bufs[b], acc_s.at[dst_v.at[c]], ssems[b], add=True)
        if with_deg:
            pltpu.async_copy(ones_v, cnt_s.at[dst_v.at[c]], osems[b],
                             add=True)

    def s_wait(b):
        pltpu.make_async_copy(bufs[b], acc_s.at[dst_v.at[0]], ssems[b]).wait()
        if with_deg:
            pltpu.make_async_copy(ones_v, cnt_s.at[dst_v.at[0]],
                                  osems[b]).wait()

    pltpu.sync_copy(zeros_hbm, acc_s.at[rows])
    pltpu.sync_copy(src_hbm.at[w], src_v)
    pltpu.sync_copy(dst_hbm.at[w], dst_v)
    if with_deg:
        pltpu.sync_copy(zeros1_hbm, cnt_s.at[rows])
        pltpu.sync_copy(ones_hbm, ones_v)
    plsc.subcore_barrier()

    for b in range(_NBUF):
        g_start(b, b)

    def round_body(r, carry):
        j = r * _NBUF
        for b in range(_NBUF):
            g_wait(b)
            s_start(b, j + b)
        for b in range(_NBUF):
            nxt = j + b + _NBUF

            @pl.when(nxt < _NCHUNK)
            def _(b=b, nxt=nxt):
                s_wait(b)
                g_start(b, nxt)
        return carry

    lax.fori_loop(0, _NCHUNK // _NBUF, round_body, 0)
    for b in range(_NBUF):
        s_wait(b)
    plsc.subcore_barrier()

    pltpu.sync_copy(acc_s.at[rows], s_hbm.at[cid, rows])
    if with_deg:
        pltpu.sync_copy(cnt_s.at[rows], cnt_hbm.at[cid, rows])


_NBUF2 = 8
_NGRP = _NCHUNK // _NBUF2


def _sc_agg_body2(tbl_hbm, src_hbm, dst_hbm, zeros_hbm, s_hbm,
                  src_v, dst_v, *rest):
    bufs = rest[:_NBUF2]
    acc_s = rest[_NBUF2]
    gsems = rest[_NBUF2 + 1:2 * _NBUF2 + 1]
    ssems = rest[2 * _NBUF2 + 1:3 * _NBUF2 + 1]
    isrc = rest[3 * _NBUF2 + 1:3 * _NBUF2 + 3]
    idst = rest[3 * _NBUF2 + 3:]
    cid = lax.axis_index("c")
    sid = lax.axis_index("s")
    w = cid * _NS + sid
    rows = pl.ds(sid * _RPT, _RPT)

    def stage(grp, slot):
        pltpu.async_copy(src_hbm.at[w, pl.ds(grp * _NBUF2, _NBUF2)],
                         src_v.at[slot], isrc[slot])
        pltpu.async_copy(dst_hbm.at[w, pl.ds(grp * _NBUF2, _NBUF2)],
                         dst_v.at[slot], idst[slot])

    def stage_wait(slot):
        pltpu.make_async_copy(src_hbm.at[w, pl.ds(0, _NBUF2)],
                              src_v.at[slot], isrc[slot]).wait()
        pltpu.make_async_copy(dst_hbm.at[w, pl.ds(0, _NBUF2)],
                              dst_v.at[slot], idst[slot]).wait()

    def g_start(b, slot):
        pltpu.async_copy(tbl_hbm.at[src_v.at[slot, b]], bufs[b], gsems[b])

    def g_wait(b):
        pltpu.make_async_copy(tbl_hbm.at[src_v.at[0, 0]], bufs[b],
                              gsems[b]).wait()

    def s_start(b, slot):
        pltpu.async_copy(bufs[b], acc_s.at[dst_v.at[slot, b]], ssems[b],
                         add=True)

    def s_wait(b):
        pltpu.make_async_copy(bufs[b], acc_s.at[dst_v.at[0, 0]],
                              ssems[b]).wait()

    pltpu.sync_copy(zeros_hbm, acc_s.at[rows])
    stage(0, 0)
    stage(1, 1)
    plsc.subcore_barrier()
    stage_wait(0)
    for b in range(_NBUF2):
        g_start(b, 0)

    def do_round(r, slot, nslot):
        for b in range(_NBUF2):
            g_wait(b)
            s_start(b, slot)

        @pl.when(r + 1 < _NGRP)
        def _():
            stage_wait(nslot)
            for b in range(_NBUF2):
                s_wait(b)
                g_start(b, nslot)

        @pl.when(r + 2 < _NGRP)
        def _():
            stage(r + 2, slot)

    def pair_body(p, carry):
        do_round(2 * p, 0, 1)
        do_round(2 * p + 1, 1, 0)
        return carry

    lax.fori_loop(0, _NGRP // 2, pair_body, 0)
    for b in range(_NBUF2):
        s_wait(b)
    plsc.subcore_barrier()
    pltpu.sync_copy(acc_s.at[rows], s_hbm.at[cid, rows])


@functools.cache
def _make_sc_agg(with_deg):
    if not with_deg:
        scratch = [
            pltpu.VMEM((2, _NBUF2, _CHUNK), jnp.int32),
            pltpu.VMEM((2, _NBUF2, _CHUNK), jnp.int32),
        ]
        scratch += [pltpu.VMEM((_CHUNK, _H), jnp.bfloat16)] * _NBUF2
        scratch.append(pltpu.VMEM_SHARED((_NPAD, _H), jnp.bfloat16))
        scratch += [pltpu.SemaphoreType.DMA] * (2 * _NBUF2 + 4)
        mesh = plsc.VectorSubcoreMesh(core_axis_name="c",
                                      subcore_axis_name="s")
        return pl.kernel(
            _sc_agg_body2,
            out_type=(jax.ShapeDtypeStruct((_NC, _NPAD, _H), jnp.bfloat16),),
            mesh=mesh,
            scratch_types=scratch,
            compiler_params=pltpu.CompilerParams(use_tc_tiling_on_sc=False),
            name="sc_agg",
        )
    scratch = [
        pltpu.VMEM((_NCHUNK, _CHUNK), jnp.int32),
        pltpu.VMEM((_NCHUNK, _CHUNK), jnp.int32),
    ]
    scratch += [pltpu.VMEM((_CHUNK, _H), jnp.bfloat16)] * _NBUF
    out_type = [jax.ShapeDtypeStruct((_NC, _NPAD, _H), jnp.bfloat16)]
    if with_deg:
        scratch.append(pltpu.VMEM((_CHUNK, _CW), jnp.float32))
        out_type.append(jax.ShapeDtypeStruct((_NC, _NPAD, _CW), jnp.float32))
    scratch += [
        pltpu.VMEM_SHARED((_NPAD, _H), jnp.bfloat16),
    ]
    if with_deg:
        scratch.append(pltpu.VMEM_SHARED((_NPAD, _CW), jnp.float32))
    scratch += [pltpu.SemaphoreType.DMA] * (2 * _NBUF)
    if with_deg:
        scratch += [pltpu.SemaphoreType.DMA] * _NBUF
    mesh = plsc.VectorSubcoreMesh(core_axis_name="c", subcore_axis_name="s")
    return pl.kernel(
        functools.partial(_sc_agg_body, with_deg),
        out_type=tuple(out_type),
        mesh=mesh,
        scratch_types=scratch,
        compiler_params=pltpu.CompilerParams(use_tc_tiling_on_sc=False),
        name="sc_agg_deg" if with_deg else "sc_agg",
    )


def _sc_agg_deg(*args):
    return _make_sc_agg(True)(*args)


def _sc_agg(*args):
    return _make_sc_agg(False)(*args)


def _tc_pre_body(x_ref, wl_ref, wr_ref, bl_ref, y_ref, r_ref):
    x = x_ref[...]
    y_ref[...] = _dotT(x, wl_ref[...]).astype(jnp.bfloat16)
    r_ref[...] = _dotT(x, wr_ref[...]) + bl_ref[...]


def _tc_pre(x_pad, wl, wr, bl):
    return pl.pallas_call(
        _tc_pre_body,
        grid=(_GRID,),
        in_specs=[
            pl.BlockSpec((_R, _DIN), lambda i: (i, 0)),
            pl.BlockSpec((_H, _DIN), lambda i: (0, 0)),
            pl.BlockSpec((_H, _DIN), lambda i: (0, 0)),
            pl.BlockSpec((1, _H), lambda i: (0, 0)),
        ],
        out_specs=[
            pl.BlockSpec((_R, _H), lambda i: (i, 0)),
            pl.BlockSpec((_R, _H), lambda i: (i, 0)),
        ],
        out_shape=[
            jax.ShapeDtypeStruct((_NPAD, _H), jnp.bfloat16),
            jax.ShapeDtypeStruct((_NPAD, _H), jnp.float32),
        ],
    )(x_pad, wl, wr, bl)


def _layer_post(s, inv, r, g, b):
    t = (s[0].astype(jnp.float32) + s[1].astype(jnp.float32)) * inv + r
    mu = jnp.mean(t, axis=-1, keepdims=True)
    d = t - mu
    var = jnp.mean(d * d, axis=-1, keepdims=True)
    hn = d * lax.rsqrt(var + 1e-5) * g + b
    return jnp.maximum(hn, 0.0)


def _tc_inv_body(cnt_ref, inv_ref):
    c = cnt_ref[0, :, :1] + cnt_ref[1, :, :1]
    inv = 1.0 / jnp.maximum(c, 1.0)
    inv_ref[...] = jnp.broadcast_to(inv, (_R, _H))


def _tc_inv(cnt):
    return pl.pallas_call(
        _tc_inv_body,
        grid=(_GRID,),
        in_specs=[pl.BlockSpec((_NC, _R, _CW), lambda i: (0, i, 0))],
        out_specs=pl.BlockSpec((_R, _H), lambda i: (i, 0)),
        out_shape=jax.ShapeDtypeStruct((_NPAD, _H), jnp.float32),
    )(cnt)


def _tc_mid_body(s_ref, cnt_ref, r_ref, g_ref, b_ref, wl_ref, bl_ref,
                 wr_ref, y_ref, r2_ref):
    h = _layer_post(s_ref[...], cnt_ref[...], r_ref[...], g_ref[...],
                    b_ref[...])
    y_ref[...] = _dotT(h, wl_ref[...]).astype(jnp.bfloat16)
    r2_ref[...] = _dotT(h, wr_ref[...]) + bl_ref[...]


def _tc_mid(s, cnt, r, g, b, wl, bl, wr):
    return pl.pallas_call(
        _tc_mid_body,
        grid=(_GRID,),
        in_specs=[
            pl.BlockSpec((_NC, _R, _H), lambda i: (0, i, 0)),
            pl.BlockSpec((_R, _H), lambda i: (i, 0)),
            pl.BlockSpec((_R, _H), lambda i: (i, 0)),
            pl.BlockSpec((1, _H), lambda i: (0, 0)),
            pl.BlockSpec((1, _H), lambda i: (0, 0)),
            pl.BlockSpec((_H, _H), lambda i: (0, 0)),
            pl.BlockSpec((1, _H), lambda i: (0, 0)),
            pl.BlockSpec((_H, _H), lambda i: (0, 0)),
        ],
        out_specs=[
            pl.BlockSpec((_R, _H), lambda i: (i, 0)),
            pl.BlockSpec((_R, _H), lambda i: (i, 0)),
        ],
        out_shape=[
            jax.ShapeDtypeStruct((_NPAD, _H), jnp.bfloat16),
            jax.ShapeDtypeStruct((_NPAD, _H), jnp.float32),
        ],
    )(s, cnt, r, g, b, wl, bl, wr)


def _tc_last_body(s_ref, cnt_ref, r_ref, g_ref, b_ref, wrf_ref, blf_ref,
                  y_ref, r2_ref):
    h = _layer_post(s_ref[...], cnt_ref[...], r_ref[...], g_ref[...],
                    b_ref[...])
    y_ref[...] = h.astype(jnp.bfloat16)
    r2_ref[...] = _dotT(h, wrf_ref[...]) + blf_ref[...]


def _tc_last(s, cnt, r, g, b, wrf, blf):
    return pl.pallas_call(
        _tc_last_body,
        grid=(_GRID,),
        in_specs=[
            pl.BlockSpec((_NC, _R, _H), lambda i: (0, i, 0)),
            pl.BlockSpec((_R, _H), lambda i: (i, 0)),
            pl.BlockSpec((_R, _H), lambda i: (i, 0)),
            pl.BlockSpec((1, _H), lambda i: (0, 0)),
            pl.BlockSpec((1, _H), lambda i: (0, 0)),
            pl.BlockSpec((_DOUT, _H), lambda i: (0, 0)),
            pl.BlockSpec((1, _DOUT), lambda i: (0, 0)),
        ],
        out_specs=[
            pl.BlockSpec((_R, _H), lambda i: (i, 0)),
            pl.BlockSpec((_R, _DOUT), lambda i: (i, 0)),
        ],
        out_shape=[
            jax.ShapeDtypeStruct((_NPAD, _H), jnp.bfloat16),
            jax.ShapeDtypeStruct((_NPAD, _DOUT), jnp.float32),
        ],
    )(s, cnt, r, g, b, wrf, blf)


def _tc_final_body(s_ref, cnt_ref, r_ref, wlf_ref, batch_ref, wo_ref, bo_ref,
                   out_ref, acc, accc):
    step = pl.program_id(0)
    s = s_ref[...].astype(jnp.float32)
    mean = (s[0] + s[1]) * cnt_ref[...]
    hf = _dotT(mean, wlf_ref[...]) + r_ref[...]
    bt = batch_ref[...]
    oh = (bt == lax.broadcasted_iota(jnp.int32, (_R, _NG), 1))
    oh = oh.astype(jnp.float32)
    part = lax.dot_general(oh, hf, (((0,), (0,)), ((), ())),
                           preferred_element_type=jnp.float32)
    pc = lax.dot_general(oh, jnp.ones((_R, 1), jnp.float32),
                         (((0,), (0,)), ((), ())),
                         preferred_element_type=jnp.float32)

    @pl.when(step == 0)
    def _():
        acc[...] = jnp.zeros_like(acc)
        accc[...] = jnp.zeros_like(accc)

    acc[...] += part
    accc[...] += pc

    @pl.when(step == pl.num_programs(0) - 1)
    def _():
        gm = acc[...] / jnp.maximum(accc[...], 1.0)
        out_ref[...] = _dotT(gm, wo_ref[...]) + bo_ref[...]


def _tc_final(s, cnt, r, wlf, batch2d, wo, bo):
    return pl.pallas_call(
        _tc_final_body,
        grid=(_GRID,),
        in_specs=[
            pl.BlockSpec((_NC, _R, _H), lambda i: (0, i, 0)),
            pl.BlockSpec((_R, _H), lambda i: (i, 0)),
            pl.BlockSpec((_R, _DOUT), lambda i: (i, 0)),
            pl.BlockSpec((_DOUT, _H), lambda i: (0, 0)),
            pl.BlockSpec((_R, 1), lambda i: (i, 0)),
            pl.BlockSpec((_DOUT, _DOUT), lambda i: (0, 0)),
            pl.BlockSpec((1, _DOUT), lambda i: (0, 0)),
        ],
        out_specs=pl.BlockSpec((_NG, _DOUT), lambda i: (0, 0)),
        out_shape=jax.ShapeDtypeStruct((_NG, _DOUT), jnp.float32),
        scratch_shapes=[
            pltpu.VMEM((_NG, _DOUT), jnp.float32),
            pltpu.VMEM((_NG, 1), jnp.float32),
        ],
    )(s, cnt, r, wlf, batch2d, wo, bo)


def kernel(x, edge_index, batch, params):
    f32 = jnp.float32
    x_pad = jnp.zeros((_NPAD, _DIN), f32).at[:_N].set(x)

    npad_rows = _NPAD - _N
    pad_idx = _N + (jnp.arange(_EPAD - _E, dtype=jnp.int32) % npad_rows)
    src = jnp.concatenate([edge_index[0], pad_idx]).reshape(_NW, _NCHUNK, _CHUNK)
    dst = jnp.concatenate([edge_index[1], pad_idx]).reshape(_NW, _NCHUNK, _CHUNK)

    zeros64 = jnp.zeros((_RPT, _H), jnp.bfloat16)
    zeros1 = jnp.zeros((_RPT, _CW), f32)
    ones1 = jnp.ones((_CHUNK, _CW), f32)
    batch2d = jnp.full((_NPAD, 1), _NG, jnp.int32).at[:_N, 0].set(batch)

    def w2(name):
        return params[name]

    def b2(name):
        return params[name].reshape(1, -1)

    y, r = _tc_pre(x_pad, w2("Wl0"), w2("Wr0"), b2("bl0"))
    s, cnt = _sc_agg_deg(y, src, dst, zeros64, ones1, zeros1)
    cnt = _tc_inv(cnt)
    for i in range(1, 5):
        y, r = _tc_mid(s, cnt, r, b2(f"g{i-1}"), b2(f"b{i-1}"),
                       w2(f"Wl{i}"), b2(f"bl{i}"), w2(f"Wr{i}"))
        (s,) = _sc_agg(y, src, dst, zeros64)
    y, r = _tc_last(s, cnt, r, b2("g4"), b2("b4"), w2("Wrf"), b2("blf"))
    (s,) = _sc_agg(y, src, dst, zeros64)
    out = _tc_final(s, cnt, r, w2("Wlf"), batch2d, w2("Wo"), b2("bo"))
    return out


# speedup vs baseline: 21.2532x; 1.0311x over previous
import functools

import jax
import jax.numpy as jnp
from jax import lax
from jax.experimental import pallas as pl
from jax.experimental.pallas import tpu as pltpu
from jax.experimental.pallas import tpu_sc as plsc

_N = 10000
_E = 320000
_DIN = 128
_H = 64
_DOUT = 128
_NG = 16

_NC = 2
_NS = 16
_NW = _NC * _NS
_NPAD = 10240
_RPT = _NPAD // _NS
_CHUNK = 128
_EW = 10240
_NCHUNK = _EW // _CHUNK
_EPAD = _NW * _EW

_CW = 16
_R = 2048
_GRID = _NPAD // _R


def _dotT(a, w):
    return lax.dot_general(a, w, (((1,), (1,)), ((), ())),
                           preferred_element_type=jnp.float32)


_NBUF = 8


def _sc_agg_body(with_deg, *refs):
    if with_deg:
        (tbl_hbm, src_hbm, dst_hbm, zeros_hbm, ones_hbm, zeros1_hbm,
         s_hbm, cnt_hbm,
         src_v, dst_v, *rest) = refs
        bufs = rest[:_NBUF]
        ones_v, acc_s, cnt_s = rest[_NBUF:_NBUF + 3]
        gsems = rest[_NBUF + 3:2 * _NBUF + 3]
        ssems = rest[2 * _NBUF + 3:3 * _NBUF + 3]
        osems = rest[3 * _NBUF + 3:]
    else:
        (tbl_hbm, src_hbm, dst_hbm, zeros_hbm,
         s_hbm,
         src_v, dst_v, *rest) = refs
        bufs = rest[:_NBUF]
        acc_s = rest[_NBUF]
        gsems = rest[_NBUF + 1:2 * _NBUF + 1]
        ssems = rest[2 * _NBUF + 1:]
    cid = lax.axis_index("c")
    sid = lax.axis_index("s")
    w = cid * _NS + sid
    rbase = sid * _RPT
    rows = pl.ds(rbase, _RPT)

    def g_start(b, c):
        pltpu.async_copy(tbl_hbm.at[src_v.at[c]], bufs[b], gsems[b])

    def g_wait(b):
        pltpu.make_async_copy(tbl_hbm.at[src_v.at[0]], bufs[b],
                              gsems[b]).wait()

    def s_start(b, c):
        pltpu.async_copy(bufs[b], acc_s.at[dst_v.at[c]], ssems[b], add=True)
        if with_deg:
            pltpu.async_copy(ones_v, cnt_s.at[dst_v.at[c]], osems[b],
                             add=True)

    def s_wait(b):
        pltpu.make_async_copy(bufs[b], acc_s.at[dst_v.at[0]], ssems[b]).wait()
        if with_deg:
            pltpu.make_async_copy(ones_v, cnt_s.at[dst_v.at[0]],
                                  osems[b]).wait()

    pltpu.sync_copy(zeros_hbm, acc_s.at[rows])
    pltpu.sync_copy(src_hbm.at[w], src_v)
    pltpu.sync_copy(dst_hbm.at[w], dst_v)
    if with_deg:
        pltpu.sync_copy(zeros1_hbm, cnt_s.at[rows])
        pltpu.sync_copy(ones_hbm, ones_v)
    plsc.subcore_barrier()

    for b in range(_NBUF):
        g_start(b, b)

    def round_body(r, carry):
        j = r * _NBUF
        for b in range(_NBUF):
            g_wait(b)
            s_start(b, j + b)
        for b in range(_NBUF):
            nxt = j + b + _NBUF

            @pl.when(nxt < _NCHUNK)
            def _(b=b, nxt=nxt):
                s_wait(b)
                g_start(b, nxt)
        return carry

    lax.fori_loop(0, _NCHUNK // _NBUF, round_body, 0)
    for b in range(_NBUF):
        s_wait(b)
    plsc.subcore_barrier()

    pltpu.sync_copy(acc_s.at[rows], s_hbm.at[cid, rows])
    if with_deg:
        pltpu.sync_copy(cnt_s.at[rows], cnt_hbm.at[cid, rows])


_NBUF2 = 8
_NGRP = _NCHUNK // _NBUF2


def _sc_agg_body2(tbl_hbm, src_hbm, dst_hbm, zeros_hbm, s_hbm,
                  src_v, dst_v, *rest):
    bufs = rest[:_NBUF2]
    acc_s = rest[_NBUF2]
    gsems = rest[_NBUF2 + 1:2 * _NBUF2 + 1]
    ssems = rest[2 * _NBUF2 + 1:3 * _NBUF2 + 1]
    isrc = rest[3 * _NBUF2 + 1:3 * _NBUF2 + 3]
    idst = rest[3 * _NBUF2 + 3:]
    cid = lax.axis_index("c")
    sid = lax.axis_index("s")
    w = cid * _NS + sid
    rows = pl.ds(sid * _RPT, _RPT)

    def stage(grp, slot):
        pltpu.async_copy(src_hbm.at[w, pl.ds(grp * _NBUF2, _NBUF2)],
                         src_v.at[slot], isrc[slot])
        pltpu.async_copy(dst_hbm.at[w, pl.ds(grp * _NBUF2, _NBUF2)],
                         dst_v.at[slot], idst[slot])

    def stage_wait(slot):
        pltpu.make_async_copy(src_hbm.at[w, pl.ds(0, _NBUF2)],
                              src_v.at[slot], isrc[slot]).wait()
        pltpu.make_async_copy(dst_hbm.at[w, pl.ds(0, _NBUF2)],
                              dst_v.at[slot], idst[slot]).wait()

    def g_start(b, slot):
        pltpu.async_copy(tbl_hbm.at[src_v.at[slot, b]], bufs[b], gsems[b])

    def g_wait(b):
        pltpu.make_async_copy(tbl_hbm.at[src_v.at[0, 0]], bufs[b],
                              gsems[b]).wait()

    def s_start(b, slot):
        pltpu.async_copy(bufs[b], acc_s.at[dst_v.at[slot, b]], ssems[b],
                         add=True)

    def s_wait(b):
        pltpu.make_async_copy(bufs[b], acc_s.at[dst_v.at[0, 0]],
                              ssems[b]).wait()

    pltpu.sync_copy(zeros_hbm, acc_s.at[rows])
    stage(0, 0)
    stage(1, 1)
    plsc.subcore_barrier()
    stage_wait(0)
    for b in range(_NBUF2):
        g_start(b, 0)

    def do_round(r, slot, nslot):
        for b in range(_NBUF2):
            g_wait(b)
            s_start(b, slot)

        @pl.when(r + 1 < _NGRP)
        def _():
            stage_wait(nslot)
            for b in range(_NBUF2):
                s_wait(b)
                g_start(b, nslot)

        @pl.when(r + 2 < _NGRP)
        def _():
            stage(r + 2, slot)

    def pair_body(p, carry):
        do_round(2 * p, 0, 1)
        do_round(2 * p + 1, 1, 0)
        return carry

    lax.fori_loop(0, _NGRP // 2, pair_body, 0)
    for b in range(_NBUF2):
        s_wait(b)
    plsc.subcore_barrier()
    pltpu.sync_copy(acc_s.at[rows], s_hbm.at[cid, rows])


@functools.cache
def _make_sc_agg(with_deg):
    if not with_deg:
        scratch = [
            pltpu.VMEM((2, _NBUF2, _CHUNK), jnp.int32),
            pltpu.VMEM((2, _NBUF2, _CHUNK), jnp.int32),
        ]
        scratch += [pltpu.VMEM((_CHUNK, _H), jnp.bfloat16)] * _NBUF2
        scratch.append(pltpu.VMEM_SHARED((_NPAD, _H), jnp.bfloat16))
        scratch += [pltpu.SemaphoreType.DMA] * (2 * _NBUF2 + 4)
        mesh = plsc.VectorSubcoreMesh(core_axis_name="c",
                                      subcore_axis_name="s")
        return pl.kernel(
            _sc_agg_body2,
            out_type=(jax.ShapeDtypeStruct((_NC, _NPAD, _H), jnp.bfloat16),),
            mesh=mesh,
            scratch_types=scratch,
            compiler_params=pltpu.CompilerParams(use_tc_tiling_on_sc=False),
            name="sc_agg",
        )
    scratch = [
        pltpu.VMEM((_NCHUNK, _CHUNK), jnp.int32),
        pltpu.VMEM((_NCHUNK, _CHUNK), jnp.int32),
    ]
    scratch += [pltpu.VMEM((_CHUNK, _H), jnp.bfloat16)] * _NBUF
    out_type = [jax.ShapeDtypeStruct((_NC, _NPAD, _H), jnp.bfloat16)]
    if with_deg:
        scratch.append(pltpu.VMEM((_CHUNK, _CW), jnp.float32))
        out_type.append(jax.ShapeDtypeStruct((_NC, _NPAD, _CW), jnp.float32))
    scratch += [
        pltpu.VMEM_SHARED((_NPAD, _H), jnp.bfloat16),
    ]
    if with_deg:
        scratch.append(pltpu.VMEM_SHARED((_NPAD, _CW), jnp.float32))
    scratch += [pltpu.SemaphoreType.DMA] * (2 * _NBUF)
    if with_deg:
        scratch += [pltpu.SemaphoreType.DMA] * _NBUF
    mesh = plsc.VectorSubcoreMesh(core_axis_name="c", subcore_axis_name="s")
    return pl.kernel(
        functools.partial(_sc_agg_body, with_deg),
        out_type=tuple(out_type),
        mesh=mesh,
        scratch_types=scratch,
        compiler_params=pltpu.CompilerParams(use_tc_tiling_on_sc=False),
        name="sc_agg_deg" if with_deg else "sc_agg",
    )


def _sc_agg_deg(*args):
    return _make_sc_agg(True)(*args)


def _sc_agg(*args):
    return _make_sc_agg(False)(*args)


def _tc_pre_body(x_ref, wl_ref, wr_ref, bl_ref, y_ref, r_ref):
    x = x_ref[...]
    y_ref[...] = _dotT(x, wl_ref[...]).astype(jnp.bfloat16)
    r_ref[...] = (_dotT(x, wr_ref[...]) + bl_ref[...]).astype(jnp.bfloat16)


def _tc_pre(x_pad, wl, wr, bl):
    return pl.pallas_call(
        _tc_pre_body,
        grid=(_GRID,),
        in_specs=[
            pl.BlockSpec((_R, _DIN), lambda i: (i, 0)),
            pl.BlockSpec((_H, _DIN), lambda i: (0, 0)),
            pl.BlockSpec((_H, _DIN), lambda i: (0, 0)),
            pl.BlockSpec((1, _H), lambda i: (0, 0)),
        ],
        out_specs=[
            pl.BlockSpec((_R, _H), lambda i: (i, 0)),
            pl.BlockSpec((_R, _H), lambda i: (i, 0)),
        ],
        out_shape=[
            jax.ShapeDtypeStruct((_NPAD, _H), jnp.bfloat16),
            jax.ShapeDtypeStruct((_NPAD, _H), jnp.bfloat16),
        ],
    )(x_pad, wl, wr, bl)


def _layer_post(s, inv, r, g, b):
    t = ((s[0].astype(jnp.float32) + s[1].astype(jnp.float32))
         * inv.astype(jnp.float32) + r.astype(jnp.float32))
    mu = jnp.mean(t, axis=-1, keepdims=True)
    d = t - mu
    var = jnp.mean(d * d, axis=-1, keepdims=True)
    hn = d * lax.rsqrt(var + 1e-5) * g + b
    return jnp.maximum(hn, 0.0)


def _tc_inv_body(cnt_ref, inv_ref):
    c = cnt_ref[0, :, :1] + cnt_ref[1, :, :1]
    inv = 1.0 / jnp.maximum(c, 1.0)
    inv_ref[...] = jnp.broadcast_to(inv, (_R, _H)).astype(jnp.bfloat16)


def _tc_inv(cnt):
    return pl.pallas_call(
        _tc_inv_body,
        grid=(_GRID,),
        in_specs=[pl.BlockSpec((_NC, _R, _CW), lambda i: (0, i, 0))],
        out_specs=pl.BlockSpec((_R, _H), lambda i: (i, 0)),
        out_shape=jax.ShapeDtypeStruct((_NPAD, _H), jnp.bfloat16),
    )(cnt)


def _tc_mid_body(s_ref, cnt_ref, r_ref, g_ref, b_ref, wl_ref, bl_ref,
                 wr_ref, y_ref, r2_ref):
    h = _layer_post(s_ref[...], cnt_ref[...], r_ref[...], g_ref[...],
                    b_ref[...])
    y_ref[...] = _dotT(h, wl_ref[...]).astype(jnp.bfloat16)
    r2_ref[...] = (_dotT(h, wr_ref[...]) + bl_ref[...]).astype(jnp.bfloat16)


def _tc_mid(s, cnt, r, g, b, wl, bl, wr):
    return pl.pallas_call(
        _tc_mid_body,
        grid=(_GRID,),
        in_specs=[
            pl.BlockSpec((_NC, _R, _H), lambda i: (0, i, 0)),
            pl.BlockSpec((_R, _H), lambda i: (i, 0)),
            pl.BlockSpec((_R, _H), lambda i: (i, 0)),
            pl.BlockSpec((1, _H), lambda i: (0, 0)),
            pl.BlockSpec((1, _H), lambda i: (0, 0)),
            pl.BlockSpec((_H, _H), lambda i: (0, 0)),
            pl.BlockSpec((1, _H), lambda i: (0, 0)),
            pl.BlockSpec((_H, _H), lambda i: (0, 0)),
        ],
        out_specs=[
            pl.BlockSpec((_R, _H), lambda i: (i, 0)),
            pl.BlockSpec((_R, _H), lambda i: (i, 0)),
        ],
        out_shape=[
            jax.ShapeDtypeStruct((_NPAD, _H), jnp.bfloat16),
            jax.ShapeDtypeStruct((_NPAD, _H), jnp.bfloat16),
        ],
    )(s, cnt, r, g, b, wl, bl, wr)


def _tc_last_body(s_ref, cnt_ref, r_ref, g_ref, b_ref, wrf_ref, blf_ref,
                  y_ref, r2_ref):
    h = _layer_post(s_ref[...], cnt_ref[...], r_ref[...], g_ref[...],
                    b_ref[...])
    y_ref[...] = h.astype(jnp.bfloat16)
    r2_ref[...] = (_dotT(h, wrf_ref[...]) + blf_ref[...]).astype(jnp.bfloat16)


def _tc_last(s, cnt, r, g, b, wrf, blf):
    return pl.pallas_call(
        _tc_last_body,
        grid=(_GRID,),
        in_specs=[
            pl.BlockSpec((_NC, _R, _H), lambda i: (0, i, 0)),
            pl.BlockSpec((_R, _H), lambda i: (i, 0)),
            pl.BlockSpec((_R, _H), lambda i: (i, 0)),
            pl.BlockSpec((1, _H), lambda i: (0, 0)),
            pl.BlockSpec((1, _H), lambda i: (0, 0)),
            pl.BlockSpec((_DOUT, _H), lambda i: (0, 0)),
            pl.BlockSpec((1, _DOUT), lambda i: (0, 0)),
        ],
        out_specs=[
            pl.BlockSpec((_R, _H), lambda i: (i, 0)),
            pl.BlockSpec((_R, _DOUT), lambda i: (i, 0)),
        ],
        out_shape=[
            jax.ShapeDtypeStruct((_NPAD, _H), jnp.bfloat16),
            jax.ShapeDtypeStruct((_NPAD, _DOUT), jnp.bfloat16),
        ],
    )(s, cnt, r, g, b, wrf, blf)


def _tc_final_body(s_ref, cnt_ref, r_ref, wlf_ref, batch_ref, wo_ref, bo_ref,
                   out_ref, acc, accc):
    step = pl.program_id(0)
    s = s_ref[...].astype(jnp.float32)
    mean = (s[0] + s[1]) * cnt_ref[...].astype(jnp.float32)
    hf = _dotT(mean, wlf_ref[...]) + r_ref[...].astype(jnp.float32)
    bt = batch_ref[...]
    oh = (bt == lax.broadcasted_iota(jnp.int32, (_R, _NG), 1))
    oh = oh.astype(jnp.float32)
    part = lax.dot_general(oh, hf, (((0,), (0,)), ((), ())),
                           preferred_element_type=jnp.float32)
    pc = lax.dot_general(oh, jnp.ones((_R, 1), jnp.float32),
                         (((0,), (0,)), ((), ())),
                         preferred_element_type=jnp.float32)

    @pl.when(step == 0)
    def _():
        acc[...] = jnp.zeros_like(acc)
        accc[...] = jnp.zeros_like(accc)

    acc[...] += part
    accc[...] += pc

    @pl.when(step == pl.num_programs(0) - 1)
    def _():
        gm = acc[...] / jnp.maximum(accc[...], 1.0)
        out_ref[...] = _dotT(gm, wo_ref[...]) + bo_ref[...]


def _tc_final(s, cnt, r, wlf, batch2d, wo, bo):
    return pl.pallas_call(
        _tc_final_body,
        grid=(_GRID,),
        in_specs=[
            pl.BlockSpec((_NC, _R, _H), lambda i: (0, i, 0)),
            pl.BlockSpec((_R, _H), lambda i: (i, 0)),
            pl.BlockSpec((_R, _DOUT), lambda i: (i, 0)),
            pl.BlockSpec((_DOUT, _H), lambda i: (0, 0)),
            pl.BlockSpec((_R, 1), lambda i: (i, 0)),
            pl.BlockSpec((_DOUT, _DOUT), lambda i: (0, 0)),
            pl.BlockSpec((1, _DOUT), lambda i: (0, 0)),
        ],
        out_specs=pl.BlockSpec((_NG, _DOUT), lambda i: (0, 0)),
        out_shape=jax.ShapeDtypeStruct((_NG, _DOUT), jnp.float32),
        scratch_shapes=[
            pltpu.VMEM((_NG, _DOUT), jnp.float32),
            pltpu.VMEM((_NG, 1), jnp.float32),
        ],
    )(s, cnt, r, wlf, batch2d, wo, bo)


def kernel(x, edge_index, batch, params):
    f32 = jnp.float32
    x_pad = jnp.zeros((_NPAD, _DIN), f32).at[:_N].set(x)

    npad_rows = _NPAD - _N
    pad_idx = _N + (jnp.arange(_EPAD - _E, dtype=jnp.int32) % npad_rows)
    src = jnp.concatenate([edge_index[0], pad_idx]).reshape(_NW, _NCHUNK, _CHUNK)
    dst = jnp.concatenate([edge_index[1], pad_idx]).reshape(_NW, _NCHUNK, _CHUNK)

    zeros64 = jnp.zeros((_RPT, _H), jnp.bfloat16)
    zeros1 = jnp.zeros((_RPT, _CW), f32)
    ones1 = jnp.ones((_CHUNK, _CW), f32)
    batch2d = jnp.full((_NPAD, 1), _NG, jnp.int32).at[:_N, 0].set(batch)

    def w2(name):
        return params[name]

    def b2(name):
        return params[name].reshape(1, -1)

    y, r = _tc_pre(x_pad, w2("Wl0"), w2("Wr0"), b2("bl0"))
    s, cnt = _sc_agg_deg(y, src, dst, zeros64, ones1, zeros1)
    cnt = _tc_inv(cnt)
    for i in range(1, 5):
        y, r = _tc_mid(s, cnt, r, b2(f"g{i-1}"), b2(f"b{i-1}"),
                       w2(f"Wl{i}"), b2(f"bl{i}"), w2(f"Wr{i}"))
        (s,) = _sc_agg(y, src, dst, zeros64)
    y, r = _tc_last(s, cnt, r, b2("g4"), b2("b4"), w2("Wrf"), b2("blf"))
    (s,) = _sc_agg(y, src, dst, zeros64)
    out = _tc_final(s, cnt, r, w2("Wlf"), batch2d, w2("Wo"), b2("bo"))
    return out


# speedup vs baseline: 21.4779x; 1.0106x over previous
import functools

import jax
import jax.numpy as jnp
from jax import lax
from jax.experimental import pallas as pl
from jax.experimental.pallas import tpu as pltpu
from jax.experimental.pallas import tpu_sc as plsc

_N = 10000
_E = 320000
_DIN = 128
_H = 64
_DOUT = 128
_NG = 16

_NC = 2
_NS = 16
_NW = _NC * _NS
_NPAD = 10240
_RPT = _NPAD // _NS
_CHUNK = 256
_EW = 10240
_NCHUNK = _EW // _CHUNK
_EPAD = _NW * _EW

_CW = 16
_R = 2048
_GRID = _NPAD // _R


def _dotT(a, w):
    return lax.dot_general(a, w, (((1,), (1,)), ((), ())),
                           preferred_element_type=jnp.float32)


_NBUF = 5


def _sc_agg_body(with_deg, *refs):
    if with_deg:
        (tbl_hbm, src_hbm, dst_hbm, zeros_hbm, ones_hbm, zeros1_hbm,
         s_hbm, cnt_hbm,
         src_v, dst_v, *rest) = refs
        bufs = rest[:_NBUF]
        ones_v, acc_s, cnt_s = rest[_NBUF:_NBUF + 3]
        gsems = rest[_NBUF + 3:2 * _NBUF + 3]
        ssems = rest[2 * _NBUF + 3:3 * _NBUF + 3]
        osems = rest[3 * _NBUF + 3:]
    else:
        (tbl_hbm, src_hbm, dst_hbm, zeros_hbm,
         s_hbm,
         src_v, dst_v, *rest) = refs
        bufs = rest[:_NBUF]
        acc_s = rest[_NBUF]
        gsems = rest[_NBUF + 1:2 * _NBUF + 1]
        ssems = rest[2 * _NBUF + 1:]
    cid = lax.axis_index("c")
    sid = lax.axis_index("s")
    w = cid * _NS + sid
    rbase = sid * _RPT
    rows = pl.ds(rbase, _RPT)

    def g_start(b, c):
        pltpu.async_copy(tbl_hbm.at[src_v.at[c]], bufs[b], gsems[b])

    def g_wait(b):
        pltpu.make_async_copy(tbl_hbm.at[src_v.at[0]], bufs[b],
                              gsems[b]).wait()

    def s_start(b, c):
        pltpu.async_copy(bufs[b], acc_s.at[dst_v.at[c]], ssems[b], add=True)
        if with_deg:
            pltpu.async_copy(ones_v, cnt_s.at[dst_v.at[c]], osems[b],
                             add=True)

    def s_wait(b):
        pltpu.make_async_copy(bufs[b], acc_s.at[dst_v.at[0]], ssems[b]).wait()
        if with_deg:
            pltpu.make_async_copy(ones_v, cnt_s.at[dst_v.at[0]],
                                  osems[b]).wait()

    pltpu.sync_copy(zeros_hbm, acc_s.at[rows])
    pltpu.sync_copy(src_hbm.at[w], src_v)
    pltpu.sync_copy(dst_hbm.at[w], dst_v)
    if with_deg:
        pltpu.sync_copy(zeros1_hbm, cnt_s.at[rows])
        pltpu.sync_copy(ones_hbm, ones_v)
    plsc.subcore_barrier()

    for b in range(_NBUF):
        g_start(b, b)

    def round_body(r, carry):
        j = r * _NBUF
        for b in range(_NBUF):
            g_wait(b)
            s_start(b, j + b)
        for b in range(_NBUF):
            nxt = j + b + _NBUF

            @pl.when(nxt < _NCHUNK)
            def _(b=b, nxt=nxt):
                s_wait(b)
                g_start(b, nxt)
        return carry

    lax.fori_loop(0, _NCHUNK // _NBUF, round_body, 0)
    for b in range(_NBUF):
        s_wait(b)
    plsc.subcore_barrier()

    pltpu.sync_copy(acc_s.at[rows], s_hbm.at[cid, rows])
    if with_deg:
        pltpu.sync_copy(cnt_s.at[rows], cnt_hbm.at[cid, rows])


_NBUF2 = 4
_NGRP = _NCHUNK // _NBUF2


def _sc_agg_body2(tbl_hbm, src_hbm, dst_hbm, zeros_hbm, s_hbm,
                  src_v, dst_v, *rest):
    bufs = rest[:_NBUF2]
    acc_s = rest[_NBUF2]
    gsems = rest[_NBUF2 + 1:2 * _NBUF2 + 1]
    ssems = rest[2 * _NBUF2 + 1:3 * _NBUF2 + 1]
    isrc = rest[3 * _NBUF2 + 1:3 * _NBUF2 + 3]
    idst = rest[3 * _NBUF2 + 3:]
    cid = lax.axis_index("c")
    sid = lax.axis_index("s")
    w = cid * _NS + sid
    rows = pl.ds(sid * _RPT, _RPT)

    def stage(grp, slot):
        pltpu.async_copy(src_hbm.at[w, pl.ds(grp * _NBUF2, _NBUF2)],
                         src_v.at[slot], isrc[slot])
        pltpu.async_copy(dst_hbm.at[w, pl.ds(grp * _NBUF2, _NBUF2)],
                         dst_v.at[slot], idst[slot])

    def stage_wait(slot):
        pltpu.make_async_copy(src_hbm.at[w, pl.ds(0, _NBUF2)],
                              src_v.at[slot], isrc[slot]).wait()
        pltpu.make_async_copy(dst_hbm.at[w, pl.ds(0, _NBUF2)],
                              dst_v.at[slot], idst[slot]).wait()

    def g_start(b, slot):
        pltpu.async_copy(tbl_hbm.at[src_v.at[slot, b]], bufs[b], gsems[b])

    def g_wait(b):
        pltpu.make_async_copy(tbl_hbm.at[src_v.at[0, 0]], bufs[b],
                              gsems[b]).wait()

    def s_start(b, slot):
        pltpu.async_copy(bufs[b], acc_s.at[dst_v.at[slot, b]], ssems[b],
                         add=True)

    def s_wait(b):
        pltpu.make_async_copy(bufs[b], acc_s.at[dst_v.at[0, 0]],
                              ssems[b]).wait()

    pltpu.sync_copy(zeros_hbm, acc_s.at[rows])
    stage(0, 0)
    stage(1, 1)
    plsc.subcore_barrier()
    stage_wait(0)
    for b in range(_NBUF2):
        g_start(b, 0)

    def do_round(r, slot, nslot):
        for b in range(_NBUF2):
            g_wait(b)
            s_start(b, slot)

        @pl.when(r + 1 < _NGRP)
        def _():
            stage_wait(nslot)
            for b in range(_NBUF2):
                s_wait(b)
                g_start(b, nslot)

        @pl.when(r + 2 < _NGRP)
        def _():
            stage(r + 2, slot)

    def pair_body(p, carry):
        do_round(2 * p, 0, 1)
        do_round(2 * p + 1, 1, 0)
        return carry

    lax.fori_loop(0, _NGRP // 2, pair_body, 0)
    for b in range(_NBUF2):
        s_wait(b)
    plsc.subcore_barrier()
    pltpu.sync_copy(acc_s.at[rows], s_hbm.at[cid, rows])


@functools.cache
def _make_sc_agg(with_deg):
    if not with_deg:
        scratch = [
            pltpu.VMEM((2, _NBUF2, _CHUNK), jnp.int32),
            pltpu.VMEM((2, _NBUF2, _CHUNK), jnp.int32),
        ]
        scratch += [pltpu.VMEM((_CHUNK, _H), jnp.bfloat16)] * _NBUF2
        scratch.append(pltpu.VMEM_SHARED((_NPAD, _H), jnp.bfloat16))
        scratch += [pltpu.SemaphoreType.DMA] * (2 * _NBUF2 + 4)
        mesh = plsc.VectorSubcoreMesh(core_axis_name="c",
                                      subcore_axis_name="s")
        return pl.kernel(
            _sc_agg_body2,
            out_type=(jax.ShapeDtypeStruct((_NC, _NPAD, _H), jnp.bfloat16),),
            mesh=mesh,
            scratch_types=scratch,
            compiler_params=pltpu.CompilerParams(use_tc_tiling_on_sc=False),
            name="sc_agg",
        )
    scratch = [
        pltpu.VMEM((_NCHUNK, _CHUNK), jnp.int32),
        pltpu.VMEM((_NCHUNK, _CHUNK), jnp.int32),
    ]
    scratch += [pltpu.VMEM((_CHUNK, _H), jnp.bfloat16)] * _NBUF
    out_type = [jax.ShapeDtypeStruct((_NC, _NPAD, _H), jnp.bfloat16)]
    if with_deg:
        scratch.append(pltpu.VMEM((_CHUNK, _CW), jnp.float32))
        out_type.append(jax.ShapeDtypeStruct((_NC, _NPAD, _CW), jnp.float32))
    scratch += [
        pltpu.VMEM_SHARED((_NPAD, _H), jnp.bfloat16),
    ]
    if with_deg:
        scratch.append(pltpu.VMEM_SHARED((_NPAD, _CW), jnp.float32))
    scratch += [pltpu.SemaphoreType.DMA] * (2 * _NBUF)
    if with_deg:
        scratch += [pltpu.SemaphoreType.DMA] * _NBUF
    mesh = plsc.VectorSubcoreMesh(core_axis_name="c", subcore_axis_name="s")
    return pl.kernel(
        functools.partial(_sc_agg_body, with_deg),
        out_type=tuple(out_type),
        mesh=mesh,
        scratch_types=scratch,
        compiler_params=pltpu.CompilerParams(use_tc_tiling_on_sc=False),
        name="sc_agg_deg" if with_deg else "sc_agg",
    )


def _sc_agg_deg(*args):
    return _make_sc_agg(True)(*args)


def _sc_agg(*args):
    return _make_sc_agg(False)(*args)


def _tc_pre_body(x_ref, wl_ref, wr_ref, bl_ref, y_ref, r_ref):
    x = x_ref[...]
    y_ref[...] = _dotT(x, wl_ref[...]).astype(jnp.bfloat16)
    r_ref[...] = (_dotT(x, wr_ref[...]) + bl_ref[...]).astype(jnp.bfloat16)


def _tc_pre(x_pad, wl, wr, bl):
    return pl.pallas_call(
        _tc_pre_body,
        grid=(_GRID,),
        in_specs=[
            pl.BlockSpec((_R, _DIN), lambda i: (i, 0)),
            pl.BlockSpec((_H, _DIN), lambda i: (0, 0)),
            pl.BlockSpec((_H, _DIN), lambda i: (0, 0)),
            pl.BlockSpec((1, _H), lambda i: (0, 0)),
        ],
        out_specs=[
            pl.BlockSpec((_R, _H), lambda i: (i, 0)),
            pl.BlockSpec((_R, _H), lambda i: (i, 0)),
        ],
        out_shape=[
            jax.ShapeDtypeStruct((_NPAD, _H), jnp.bfloat16),
            jax.ShapeDtypeStruct((_NPAD, _H), jnp.bfloat16),
        ],
    )(x_pad, wl, wr, bl)


def _layer_post(s, inv, r, g, b):
    t = ((s[0].astype(jnp.float32) + s[1].astype(jnp.float32))
         * inv.astype(jnp.float32) + r.astype(jnp.float32))
    mu = jnp.mean(t, axis=-1, keepdims=True)
    d = t - mu
    var = jnp.mean(d * d, axis=-1, keepdims=True)
    hn = d * lax.rsqrt(var + 1e-5) * g + b
    return jnp.maximum(hn, 0.0)


def _tc_inv_body(cnt_ref, inv_ref):
    c = cnt_ref[0, :, :1] + cnt_ref[1, :, :1]
    inv = 1.0 / jnp.maximum(c, 1.0)
    inv_ref[...] = jnp.broadcast_to(inv, (_R, _H)).astype(jnp.bfloat16)


def _tc_inv(cnt):
    return pl.pallas_call(
        _tc_inv_body,
        grid=(_GRID,),
        in_specs=[pl.BlockSpec((_NC, _R, _CW), lambda i: (0, i, 0))],
        out_specs=pl.BlockSpec((_R, _H), lambda i: (i, 0)),
        out_shape=jax.ShapeDtypeStruct((_NPAD, _H), jnp.bfloat16),
    )(cnt)


def _tc_mid_body(s_ref, cnt_ref, r_ref, g_ref, b_ref, wl_ref, bl_ref,
                 wr_ref, y_ref, r2_ref):
    h = _layer_post(s_ref[...], cnt_ref[...], r_ref[...], g_ref[...],
                    b_ref[...])
    y_ref[...] = _dotT(h, wl_ref[...]).astype(jnp.bfloat16)
    r2_ref[...] = (_dotT(h, wr_ref[...]) + bl_ref[...]).astype(jnp.bfloat16)


def _tc_mid(s, cnt, r, g, b, wl, bl, wr):
    return pl.pallas_call(
        _tc_mid_body,
        grid=(_GRID,),
        in_specs=[
            pl.BlockSpec((_NC, _R, _H), lambda i: (0, i, 0)),
            pl.BlockSpec((_R, _H), lambda i: (i, 0)),
            pl.BlockSpec((_R, _H), lambda i: (i, 0)),
            pl.BlockSpec((1, _H), lambda i: (0, 0)),
            pl.BlockSpec((1, _H), lambda i: (0, 0)),
            pl.BlockSpec((_H, _H), lambda i: (0, 0)),
            pl.BlockSpec((1, _H), lambda i: (0, 0)),
            pl.BlockSpec((_H, _H), lambda i: (0, 0)),
        ],
        out_specs=[
            pl.BlockSpec((_R, _H), lambda i: (i, 0)),
            pl.BlockSpec((_R, _H), lambda i: (i, 0)),
        ],
        out_shape=[
            jax.ShapeDtypeStruct((_NPAD, _H), jnp.bfloat16),
            jax.ShapeDtypeStruct((_NPAD, _H), jnp.bfloat16),
        ],
    )(s, cnt, r, g, b, wl, bl, wr)


def _tc_last_body(s_ref, cnt_ref, r_ref, g_ref, b_ref, wrf_ref, blf_ref,
                  y_ref, r2_ref):
    h = _layer_post(s_ref[...], cnt_ref[...], r_ref[...], g_ref[...],
                    b_ref[...])
    y_ref[...] = h.astype(jnp.bfloat16)
    r2_ref[...] = (_dotT(h, wrf_ref[...]) + blf_ref[...]).astype(jnp.bfloat16)


def _tc_last(s, cnt, r, g, b, wrf, blf):
    return pl.pallas_call(
        _tc_last_body,
        grid=(_GRID,),
        in_specs=[
            pl.BlockSpec((_NC, _R, _H), lambda i: (0, i, 0)),
            pl.BlockSpec((_R, _H), lambda i: (i, 0)),
            pl.BlockSpec((_R, _H), lambda i: (i, 0)),
            pl.BlockSpec((1, _H), lambda i: (0, 0)),
            pl.BlockSpec((1, _H), lambda i: (0, 0)),
            pl.BlockSpec((_DOUT, _H), lambda i: (0, 0)),
            pl.BlockSpec((1, _DOUT), lambda i: (0, 0)),
        ],
        out_specs=[
            pl.BlockSpec((_R, _H), lambda i: (i, 0)),
            pl.BlockSpec((_R, _DOUT), lambda i: (i, 0)),
        ],
        out_shape=[
            jax.ShapeDtypeStruct((_NPAD, _H), jnp.bfloat16),
            jax.ShapeDtypeStruct((_NPAD, _DOUT), jnp.bfloat16),
        ],
    )(s, cnt, r, g, b, wrf, blf)


def _tc_final_body(s_ref, cnt_ref, r_ref, wlf_ref, batch_ref, wo_ref, bo_ref,
                   out_ref, acc, accc):
    step = pl.program_id(0)
    s = s_ref[...].astype(jnp.float32)
    mean = (s[0] + s[1]) * cnt_ref[...].astype(jnp.float32)
    hf = _dotT(mean, wlf_ref[...]) + r_ref[...].astype(jnp.float32)
    bt = batch_ref[...]
    oh = (bt == lax.broadcasted_iota(jnp.int32, (_R, _NG), 1))
    oh = oh.astype(jnp.float32)
    part = lax.dot_general(oh, hf, (((0,), (0,)), ((), ())),
                           preferred_element_type=jnp.float32)
    pc = lax.dot_general(oh, jnp.ones((_R, 1), jnp.float32),
                         (((0,), (0,)), ((), ())),
                         preferred_element_type=jnp.float32)

    @pl.when(step == 0)
    def _():
        acc[...] = jnp.zeros_like(acc)
        accc[...] = jnp.zeros_like(accc)

    acc[...] += part
    accc[...] += pc

    @pl.when(step == pl.num_programs(0) - 1)
    def _():
        gm = acc[...] / jnp.maximum(accc[...], 1.0)
        out_ref[...] = _dotT(gm, wo_ref[...]) + bo_ref[...]


def _tc_final(s, cnt, r, wlf, batch2d, wo, bo):
    return pl.pallas_call(
        _tc_final_body,
        grid=(_GRID,),
        in_specs=[
            pl.BlockSpec((_NC, _R, _H), lambda i: (0, i, 0)),
            pl.BlockSpec((_R, _H), lambda i: (i, 0)),
            pl.BlockSpec((_R, _DOUT), lambda i: (i, 0)),
            pl.BlockSpec((_DOUT, _H), lambda i: (0, 0)),
            pl.BlockSpec((_R, 1), lambda i: (i, 0)),
            pl.BlockSpec((_DOUT, _DOUT), lambda i: (0, 0)),
            pl.BlockSpec((1, _DOUT), lambda i: (0, 0)),
        ],
        out_specs=pl.BlockSpec((_NG, _DOUT), lambda i: (0, 0)),
        out_shape=jax.ShapeDtypeStruct((_NG, _DOUT), jnp.float32),
        scratch_shapes=[
            pltpu.VMEM((_NG, _DOUT), jnp.float32),
            pltpu.VMEM((_NG, 1), jnp.float32),
        ],
    )(s, cnt, r, wlf, batch2d, wo, bo)


def kernel(x, edge_index, batch, params):
    f32 = jnp.float32
    x_pad = jnp.zeros((_NPAD, _DIN), f32).at[:_N].set(x)

    npad_rows = _NPAD - _N
    pad_idx = _N + (jnp.arange(_EPAD - _E, dtype=jnp.int32) % npad_rows)
    src = jnp.concatenate([edge_index[0], pad_idx]).reshape(_NW, _NCHUNK, _CHUNK)
    dst = jnp.concatenate([edge_index[1], pad_idx]).reshape(_NW, _NCHUNK, _CHUNK)

    zeros64 = jnp.zeros((_RPT, _H), jnp.bfloat16)
    zeros1 = jnp.zeros((_RPT, _CW), f32)
    ones1 = jnp.ones((_CHUNK, _CW), f32)
    batch2d = jnp.full((_NPAD, 1), _NG, jnp.int32).at[:_N, 0].set(batch)

    def w2(name):
        return params[name]

    def b2(name):
        return params[name].reshape(1, -1)

    y, r = _tc_pre(x_pad, w2("Wl0"), w2("Wr0"), b2("bl0"))
    s, cnt = _sc_agg_deg(y, src, dst, zeros64, ones1, zeros1)
    cnt = _tc_inv(cnt)
    for i in range(1, 5):
        y, r = _tc_mid(s, cnt, r, b2(f"g{i-1}"), b2(f"b{i-1}"),
                       w2(f"Wl{i}"), b2(f"bl{i}"), w2(f"Wr{i}"))
        (s,) = _sc_agg(y, src, dst, zeros64)
    y, r = _tc_last(s, cnt, r, b2("g4"), b2("b4"), w2("Wrf"), b2("blf"))
    (s,) = _sc_agg(y, src, dst, zeros64)
    out = _tc_final(s, cnt, r, w2("Wlf"), batch2d, w2("Wo"), b2("bo"))
    return out


# speedup vs baseline: 21.8538x; 1.0175x over previous
import functools

import jax
import jax.numpy as jnp
from jax import lax
from jax.experimental import pallas as pl
from jax.experimental.pallas import tpu as pltpu
from jax.experimental.pallas import tpu_sc as plsc

_N = 10000
_E = 320000
_DIN = 128
_H = 64
_DOUT = 128
_NG = 16

_NC = 2
_NS = 16
_NW = _NC * _NS
_NPAD = 10240
_RPT = _NPAD // _NS
_CHUNK = 256
_EW = 10240
_NCHUNK = _EW // _CHUNK
_EPAD = _NW * _EW

_CW = 16
_R = 2048
_GRID = _NPAD // _R


def _dotT(a, w):
    return lax.dot_general(a, w, (((1,), (1,)), ((), ())),
                           preferred_element_type=jnp.float32)


_NBUF = 5


def _sc_agg_body(with_deg, *refs):
    if with_deg:
        (tbl_hbm, src_hbm, dst_hbm, zeros_hbm, ones_hbm, zeros1_hbm,
         s_hbm, cnt_hbm,
         src_v, dst_v, *rest) = refs
        bufs = rest[:_NBUF]
        ones_v, acc_s, cnt_s = rest[_NBUF:_NBUF + 3]
        gsems = rest[_NBUF + 3:2 * _NBUF + 3]
        ssems = rest[2 * _NBUF + 3:3 * _NBUF + 3]
        osems = rest[3 * _NBUF + 3:]
    else:
        (tbl_hbm, src_hbm, dst_hbm, zeros_hbm,
         s_hbm,
         src_v, dst_v, *rest) = refs
        bufs = rest[:_NBUF]
        acc_s = rest[_NBUF]
        gsems = rest[_NBUF + 1:2 * _NBUF + 1]
        ssems = rest[2 * _NBUF + 1:]
    cid = lax.axis_index("c")
    sid = lax.axis_index("s")
    w = cid * _NS + sid
    rbase = sid * _RPT
    rows = pl.ds(rbase, _RPT)

    def g_start(b, c):
        pltpu.async_copy(tbl_hbm.at[src_v.at[c]], bufs[b], gsems[b])

    def g_wait(b):
        pltpu.make_async_copy(tbl_hbm.at[src_v.at[0]], bufs[b],
                              gsems[b]).wait()

    def s_start(b, c):
        pltpu.async_copy(bufs[b], acc_s.at[dst_v.at[c]], ssems[b], add=True)
        if with_deg:
            pltpu.async_copy(ones_v, cnt_s.at[dst_v.at[c]], osems[b],
                             add=True)

    def s_wait(b):
        pltpu.make_async_copy(bufs[b], acc_s.at[dst_v.at[0]], ssems[b]).wait()
        if with_deg:
            pltpu.make_async_copy(ones_v, cnt_s.at[dst_v.at[0]],
                                  osems[b]).wait()

    pltpu.sync_copy(zeros_hbm, acc_s.at[rows])
    pltpu.sync_copy(src_hbm.at[w], src_v)
    pltpu.sync_copy(dst_hbm.at[w], dst_v)
    if with_deg:
        pltpu.sync_copy(zeros1_hbm, cnt_s.at[rows])
        pltpu.sync_copy(ones_hbm, ones_v)
    plsc.subcore_barrier()

    for b in range(_NBUF):
        g_start(b, b)

    def round_body(r, carry):
        j = r * _NBUF
        for b in range(_NBUF):
            g_wait(b)
            s_start(b, j + b)
        for b in range(_NBUF):
            nxt = j + b + _NBUF

            @pl.when(nxt < _NCHUNK)
            def _(b=b, nxt=nxt):
                s_wait(b)
                g_start(b, nxt)
        return carry

    lax.fori_loop(0, _NCHUNK // _NBUF, round_body, 0)
    for b in range(_NBUF):
        s_wait(b)
    plsc.subcore_barrier()

    pltpu.sync_copy(acc_s.at[rows], s_hbm.at[cid, rows])
    if with_deg:
        pltpu.sync_copy(cnt_s.at[rows], cnt_hbm.at[cid, rows])


_NBUF2 = 4
_NGRP = _NCHUNK // _NBUF2


def _sc_agg_body2(tbl_hbm, src_hbm, dst_hbm, zeros_hbm, s_hbm,
                  src_v, dst_v, *rest):
    bufs = rest[:_NBUF2]
    acc_s = rest[_NBUF2]
    gsems = rest[_NBUF2 + 1:2 * _NBUF2 + 1]
    ssems = rest[2 * _NBUF2 + 1:3 * _NBUF2 + 1]
    isrc = rest[3 * _NBUF2 + 1:3 * _NBUF2 + 3]
    idst = rest[3 * _NBUF2 + 3:]
    cid = lax.axis_index("c")
    sid = lax.axis_index("s")
    w = cid * _NS + sid
    rows = pl.ds(sid * _RPT, _RPT)

    def stage(grp, slot):
        pltpu.async_copy(src_hbm.at[w, pl.ds(grp * _NBUF2, _NBUF2)],
                         src_v.at[slot], isrc[slot])
        pltpu.async_copy(dst_hbm.at[w, pl.ds(grp * _NBUF2, _NBUF2)],
                         dst_v.at[slot], idst[slot])

    def stage_wait(slot):
        pltpu.make_async_copy(src_hbm.at[w, pl.ds(0, _NBUF2)],
                              src_v.at[slot], isrc[slot]).wait()
        pltpu.make_async_copy(dst_hbm.at[w, pl.ds(0, _NBUF2)],
                              dst_v.at[slot], idst[slot]).wait()

    def g_start(b, slot):
        pltpu.async_copy(tbl_hbm.at[src_v.at[slot, b]], bufs[b], gsems[b])

    def g_wait(b):
        pltpu.make_async_copy(tbl_hbm.at[src_v.at[0, 0]], bufs[b],
                              gsems[b]).wait()

    def s_start(b, slot):
        pltpu.async_copy(bufs[b], acc_s.at[dst_v.at[slot, b]], ssems[b],
                         add=True)

    def s_wait(b):
        pltpu.make_async_copy(bufs[b], acc_s.at[dst_v.at[0, 0]],
                              ssems[b]).wait()

    pltpu.sync_copy(zeros_hbm, acc_s.at[rows])
    stage(0, 0)
    stage(1, 1)
    plsc.subcore_barrier()
    stage_wait(0)
    for b in range(_NBUF2):
        g_start(b, 0)

    def do_round(r, slot, nslot):
        for b in range(_NBUF2):
            g_wait(b)
            s_start(b, slot)

        @pl.when(r + 1 < _NGRP)
        def _():
            stage_wait(nslot)
            for b in range(_NBUF2):
                s_wait(b)
                g_start(b, nslot)

        @pl.when(r + 2 < _NGRP)
        def _():
            stage(r + 2, slot)

    def pair_body(p, carry):
        do_round(2 * p, 0, 1)
        do_round(2 * p + 1, 1, 0)
        return carry

    lax.fori_loop(0, _NGRP // 2, pair_body, 0)
    for b in range(_NBUF2):
        s_wait(b)
    plsc.subcore_barrier()
    pltpu.sync_copy(acc_s.at[rows], s_hbm.at[cid, rows])


@functools.cache
def _make_sc_agg(with_deg):
    if not with_deg:
        scratch = [
            pltpu.VMEM((2, _NBUF2, _CHUNK), jnp.int32),
            pltpu.VMEM((2, _NBUF2, _CHUNK), jnp.int32),
        ]
        scratch += [pltpu.VMEM((_CHUNK, _H), jnp.bfloat16)] * _NBUF2
        scratch.append(pltpu.VMEM_SHARED((_NPAD, _H), jnp.bfloat16))
        scratch += [pltpu.SemaphoreType.DMA] * (2 * _NBUF2 + 4)
        mesh = plsc.VectorSubcoreMesh(core_axis_name="c",
                                      subcore_axis_name="s")
        return pl.kernel(
            _sc_agg_body2,
            out_type=(jax.ShapeDtypeStruct((_NC, _NPAD, _H), jnp.bfloat16),),
            mesh=mesh,
            scratch_types=scratch,
            compiler_params=pltpu.CompilerParams(use_tc_tiling_on_sc=False),
            name="sc_agg",
        )
    scratch = [
        pltpu.VMEM((_NCHUNK, _CHUNK), jnp.int32),
        pltpu.VMEM((_NCHUNK, _CHUNK), jnp.int32),
    ]
    scratch += [pltpu.VMEM((_CHUNK, _H), jnp.bfloat16)] * _NBUF
    out_type = [jax.ShapeDtypeStruct((_NC, _NPAD, _H), jnp.bfloat16)]
    if with_deg:
        scratch.append(pltpu.VMEM((_CHUNK, _CW), jnp.bfloat16))
        out_type.append(jax.ShapeDtypeStruct((_NC, _NPAD, _CW), jnp.bfloat16))
    scratch += [
        pltpu.VMEM_SHARED((_NPAD, _H), jnp.bfloat16),
    ]
    if with_deg:
        scratch.append(pltpu.VMEM_SHARED((_NPAD, _CW), jnp.bfloat16))
    scratch += [pltpu.SemaphoreType.DMA] * (2 * _NBUF)
    if with_deg:
        scratch += [pltpu.SemaphoreType.DMA] * _NBUF
    mesh = plsc.VectorSubcoreMesh(core_axis_name="c", subcore_axis_name="s")
    return pl.kernel(
        functools.partial(_sc_agg_body, with_deg),
        out_type=tuple(out_type),
        mesh=mesh,
        scratch_types=scratch,
        compiler_params=pltpu.CompilerParams(use_tc_tiling_on_sc=False),
        name="sc_agg_deg" if with_deg else "sc_agg",
    )


def _sc_agg_deg(*args):
    return _make_sc_agg(True)(*args)


def _sc_agg(*args):
    return _make_sc_agg(False)(*args)


def _tc_pre_body(x_ref, wl_ref, wr_ref, bl_ref, y_ref, r_ref):
    x = x_ref[...]
    y_ref[...] = _dotT(x, wl_ref[...]).astype(jnp.bfloat16)
    r_ref[...] = (_dotT(x, wr_ref[...]) + bl_ref[...]).astype(jnp.bfloat16)


def _tc_pre(x_pad, wl, wr, bl):
    return pl.pallas_call(
        _tc_pre_body,
        grid=(_GRID,),
        in_specs=[
            pl.BlockSpec((_R, _DIN), lambda i: (i, 0)),
            pl.BlockSpec((_H, _DIN), lambda i: (0, 0)),
            pl.BlockSpec((_H, _DIN), lambda i: (0, 0)),
            pl.BlockSpec((1, _H), lambda i: (0, 0)),
        ],
        out_specs=[
            pl.BlockSpec((_R, _H), lambda i: (i, 0)),
            pl.BlockSpec((_R, _H), lambda i: (i, 0)),
        ],
        out_shape=[
            jax.ShapeDtypeStruct((_NPAD, _H), jnp.bfloat16),
            jax.ShapeDtypeStruct((_NPAD, _H), jnp.bfloat16),
        ],
    )(x_pad, wl, wr, bl)


def _layer_post(s, inv, r, g, b):
    t = ((s[0].astype(jnp.float32) + s[1].astype(jnp.float32))
         * inv.astype(jnp.float32) + r.astype(jnp.float32))
    mu = jnp.mean(t, axis=-1, keepdims=True)
    d = t - mu
    var = jnp.mean(d * d, axis=-1, keepdims=True)
    hn = d * lax.rsqrt(var + 1e-5) * g + b
    return jnp.maximum(hn, 0.0)


def _tc_inv_body(cnt_ref, inv_ref):
    c = (cnt_ref[0, :, :1].astype(jnp.float32)
         + cnt_ref[1, :, :1].astype(jnp.float32))
    inv = 1.0 / jnp.maximum(c, 1.0)
    inv_ref[...] = jnp.broadcast_to(inv, (_R, _H)).astype(jnp.bfloat16)


def _tc_inv(cnt):
    return pl.pallas_call(
        _tc_inv_body,
        grid=(_GRID,),
        in_specs=[pl.BlockSpec((_NC, _R, _CW), lambda i: (0, i, 0))],
        out_specs=pl.BlockSpec((_R, _H), lambda i: (i, 0)),
        out_shape=jax.ShapeDtypeStruct((_NPAD, _H), jnp.bfloat16),
    )(cnt)


def _tc_mid_body(s_ref, cnt_ref, r_ref, g_ref, b_ref, wl_ref, bl_ref,
                 wr_ref, y_ref, r2_ref):
    h = _layer_post(s_ref[...], cnt_ref[...], r_ref[...], g_ref[...],
                    b_ref[...])
    y_ref[...] = _dotT(h, wl_ref[...]).astype(jnp.bfloat16)
    r2_ref[...] = (_dotT(h, wr_ref[...]) + bl_ref[...]).astype(jnp.bfloat16)


def _tc_mid(s, cnt, r, g, b, wl, bl, wr):
    return pl.pallas_call(
        _tc_mid_body,
        grid=(_GRID,),
        in_specs=[
            pl.BlockSpec((_NC, _R, _H), lambda i: (0, i, 0)),
            pl.BlockSpec((_R, _H), lambda i: (i, 0)),
            pl.BlockSpec((_R, _H), lambda i: (i, 0)),
            pl.BlockSpec((1, _H), lambda i: (0, 0)),
            pl.BlockSpec((1, _H), lambda i: (0, 0)),
            pl.BlockSpec((_H, _H), lambda i: (0, 0)),
            pl.BlockSpec((1, _H), lambda i: (0, 0)),
            pl.BlockSpec((_H, _H), lambda i: (0, 0)),
        ],
        out_specs=[
            pl.BlockSpec((_R, _H), lambda i: (i, 0)),
            pl.BlockSpec((_R, _H), lambda i: (i, 0)),
        ],
        out_shape=[
            jax.ShapeDtypeStruct((_NPAD, _H), jnp.bfloat16),
            jax.ShapeDtypeStruct((_NPAD, _H), jnp.bfloat16),
        ],
    )(s, cnt, r, g, b, wl, bl, wr)


def _tc_last_body(s_ref, cnt_ref, r_ref, g_ref, b_ref, wrf_ref, blf_ref,
                  y_ref, r2_ref):
    h = _layer_post(s_ref[...], cnt_ref[...], r_ref[...], g_ref[...],
                    b_ref[...])
    y_ref[...] = h.astype(jnp.bfloat16)
    r2_ref[...] = (_dotT(h, wrf_ref[...]) + blf_ref[...]).astype(jnp.bfloat16)


def _tc_last(s, cnt, r, g, b, wrf, blf):
    return pl.pallas_call(
        _tc_last_body,
        grid=(_GRID,),
        in_specs=[
            pl.BlockSpec((_NC, _R, _H), lambda i: (0, i, 0)),
            pl.BlockSpec((_R, _H), lambda i: (i, 0)),
            pl.BlockSpec((_R, _H), lambda i: (i, 0)),
            pl.BlockSpec((1, _H), lambda i: (0, 0)),
            pl.BlockSpec((1, _H), lambda i: (0, 0)),
            pl.BlockSpec((_DOUT, _H), lambda i: (0, 0)),
            pl.BlockSpec((1, _DOUT), lambda i: (0, 0)),
        ],
        out_specs=[
            pl.BlockSpec((_R, _H), lambda i: (i, 0)),
            pl.BlockSpec((_R, _DOUT), lambda i: (i, 0)),
        ],
        out_shape=[
            jax.ShapeDtypeStruct((_NPAD, _H), jnp.bfloat16),
            jax.ShapeDtypeStruct((_NPAD, _DOUT), jnp.bfloat16),
        ],
    )(s, cnt, r, g, b, wrf, blf)


def _tc_final_body(s_ref, cnt_ref, r_ref, wlf_ref, batch_ref, wo_ref, bo_ref,
                   out_ref, acc, accc):
    step = pl.program_id(0)
    s = s_ref[...].astype(jnp.float32)
    mean = (s[0] + s[1]) * cnt_ref[...].astype(jnp.float32)
    hf = _dotT(mean, wlf_ref[...]) + r_ref[...].astype(jnp.float32)
    bt = batch_ref[...]
    oh = (bt == lax.broadcasted_iota(jnp.int32, (_R, _NG), 1))
    oh = oh.astype(jnp.float32)
    part = lax.dot_general(oh, hf, (((0,), (0,)), ((), ())),
                           preferred_element_type=jnp.float32)
    pc = lax.dot_general(oh, jnp.ones((_R, 1), jnp.float32),
                         (((0,), (0,)), ((), ())),
                         preferred_element_type=jnp.float32)

    @pl.when(step == 0)
    def _():
        acc[...] = jnp.zeros_like(acc)
        accc[...] = jnp.zeros_like(accc)

    acc[...] += part
    accc[...] += pc

    @pl.when(step == pl.num_programs(0) - 1)
    def _():
        gm = acc[...] / jnp.maximum(accc[...], 1.0)
        out_ref[...] = _dotT(gm, wo_ref[...]) + bo_ref[...]


def _tc_final(s, cnt, r, wlf, batch2d, wo, bo):
    return pl.pallas_call(
        _tc_final_body,
        grid=(_GRID,),
        in_specs=[
            pl.BlockSpec((_NC, _R, _H), lambda i: (0, i, 0)),
            pl.BlockSpec((_R, _H), lambda i: (i, 0)),
            pl.BlockSpec((_R, _DOUT), lambda i: (i, 0)),
            pl.BlockSpec((_DOUT, _H), lambda i: (0, 0)),
            pl.BlockSpec((_R, 1), lambda i: (i, 0)),
            pl.BlockSpec((_DOUT, _DOUT), lambda i: (0, 0)),
            pl.BlockSpec((1, _DOUT), lambda i: (0, 0)),
        ],
        out_specs=pl.BlockSpec((_NG, _DOUT), lambda i: (0, 0)),
        out_shape=jax.ShapeDtypeStruct((_NG, _DOUT), jnp.float32),
        scratch_shapes=[
            pltpu.VMEM((_NG, _DOUT), jnp.float32),
            pltpu.VMEM((_NG, 1), jnp.float32),
        ],
    )(s, cnt, r, wlf, batch2d, wo, bo)


def kernel(x, edge_index, batch, params):
    f32 = jnp.float32
    x_pad = jnp.zeros((_NPAD, _DIN), f32).at[:_N].set(x)

    npad_rows = _NPAD - _N
    pad_idx = _N + (jnp.arange(_EPAD - _E, dtype=jnp.int32) % npad_rows)
    src = jnp.concatenate([edge_index[0], pad_idx]).reshape(_NW, _NCHUNK, _CHUNK)
    dst = jnp.concatenate([edge_index[1], pad_idx]).reshape(_NW, _NCHUNK, _CHUNK)

    zeros64 = jnp.zeros((_RPT, _H), jnp.bfloat16)
    zeros1 = jnp.zeros((_RPT, _CW), jnp.bfloat16)
    ones1 = jnp.ones((_CHUNK, _CW), jnp.bfloat16)
    batch2d = jnp.full((_NPAD, 1), _NG, jnp.int32).at[:_N, 0].set(batch)

    def w2(name):
        return params[name]

    def b2(name):
        return params[name].reshape(1, -1)

    y, r = _tc_pre(x_pad, w2("Wl0"), w2("Wr0"), b2("bl0"))
    s, cnt = _sc_agg_deg(y, src, dst, zeros64, ones1, zeros1)
    cnt = _tc_inv(cnt)
    for i in range(1, 5):
        y, r = _tc_mid(s, cnt, r, b2(f"g{i-1}"), b2(f"b{i-1}"),
                       w2(f"Wl{i}"), b2(f"bl{i}"), w2(f"Wr{i}"))
        (s,) = _sc_agg(y, src, dst, zeros64)
    y, r = _tc_last(s, cnt, r, b2("g4"), b2("b4"), w2("Wrf"), b2("blf"))
    (s,) = _sc_agg(y, src, dst, zeros64)
    out = _tc_final(s, cnt, r, w2("Wlf"), batch2d, w2("Wo"), b2("bo"))
    return out


# speedup vs baseline: 21.9549x; 1.0046x over previous
import functools

import jax
import jax.numpy as jnp
from jax import lax
from jax.experimental import pallas as pl
from jax.experimental.pallas import tpu as pltpu
from jax.experimental.pallas import tpu_sc as plsc

_N = 10000
_E = 320000
_DIN = 128
_H = 64
_DOUT = 128
_NG = 16

_NC = 2
_NS = 16
_NW = _NC * _NS
_NPAD = 10240
_RPT = _NPAD // _NS
_CHUNK = 256
_EW = 10240
_NCHUNK = _EW // _CHUNK
_EPAD = _NW * _EW

_CW = 16
_R = 2048
_GRID = _NPAD // _R


def _dotT(a, w):
    return lax.dot_general(a, w, (((1,), (1,)), ((), ())),
                           preferred_element_type=jnp.float32)


_NBUF = 5


def _sc_agg_body(with_deg, *refs):
    if with_deg:
        (tbl_hbm, src_hbm, dst_hbm, zeros_hbm, ones_hbm, zeros1_hbm,
         s_hbm, cnt_hbm,
         src_v, dst_v, *rest) = refs
        bufs = rest[:_NBUF]
        ones_v, acc_s, cnt_s = rest[_NBUF:_NBUF + 3]
        gsems = rest[_NBUF + 3:2 * _NBUF + 3]
        ssems = rest[2 * _NBUF + 3:3 * _NBUF + 3]
        osems = rest[3 * _NBUF + 3:]
    else:
        (tbl_hbm, src_hbm, dst_hbm, zeros_hbm,
         s_hbm,
         src_v, dst_v, *rest) = refs
        bufs = rest[:_NBUF]
        acc_s = rest[_NBUF]
        gsems = rest[_NBUF + 1:2 * _NBUF + 1]
        ssems = rest[2 * _NBUF + 1:]
    cid = lax.axis_index("c")
    sid = lax.axis_index("s")
    w = cid * _NS + sid
    rbase = sid * _RPT
    rows = pl.ds(rbase, _RPT)

    def g_start(b, c):
        pltpu.async_copy(tbl_hbm.at[src_v.at[c]], bufs[b], gsems[b])

    def g_wait(b):
        pltpu.make_async_copy(tbl_hbm.at[src_v.at[0]], bufs[b],
                              gsems[b]).wait()

    def s_start(b, c):
        pltpu.async_copy(bufs[b], acc_s.at[dst_v.at[c]], ssems[b], add=True)
        if with_deg:
            pltpu.async_copy(ones_v, cnt_s.at[dst_v.at[c]], osems[b],
                             add=True)

    def s_wait(b):
        pltpu.make_async_copy(bufs[b], acc_s.at[dst_v.at[0]], ssems[b]).wait()
        if with_deg:
            pltpu.make_async_copy(ones_v, cnt_s.at[dst_v.at[0]],
                                  osems[b]).wait()

    pltpu.sync_copy(zeros_hbm, acc_s.at[rows])
    pltpu.sync_copy(src_hbm.at[w], src_v)
    pltpu.sync_copy(dst_hbm.at[w], dst_v)
    if with_deg:
        pltpu.sync_copy(zeros1_hbm, cnt_s.at[rows])
        pltpu.sync_copy(ones_hbm, ones_v)
    plsc.subcore_barrier()

    for b in range(_NBUF):
        g_start(b, b)

    def round_body(r, carry):
        j = r * _NBUF
        for b in range(_NBUF):
            g_wait(b)
            s_start(b, j + b)
        for b in range(_NBUF):
            nxt = j + b + _NBUF

            @pl.when(nxt < _NCHUNK)
            def _(b=b, nxt=nxt):
                s_wait(b)
                g_start(b, nxt)
        return carry

    lax.fori_loop(0, _NCHUNK // _NBUF, round_body, 0)
    for b in range(_NBUF):
        s_wait(b)
    plsc.subcore_barrier()

    pltpu.sync_copy(acc_s.at[rows], s_hbm.at[cid, rows])
    if with_deg:
        pltpu.sync_copy(cnt_s.at[rows], cnt_hbm.at[cid, rows])


_NBUF2 = 5
_NGRP = _NCHUNK // _NBUF2


def _sc_agg_body2(tbl_hbm, src_hbm, dst_hbm, zeros_hbm, s_hbm,
                  src_v, dst_v, *rest):
    bufs = rest[:_NBUF2]
    acc_s = rest[_NBUF2]
    gsems = rest[_NBUF2 + 1:2 * _NBUF2 + 1]
    ssems = rest[2 * _NBUF2 + 1:3 * _NBUF2 + 1]
    isrc = rest[3 * _NBUF2 + 1:3 * _NBUF2 + 3]
    idst = rest[3 * _NBUF2 + 3:]
    cid = lax.axis_index("c")
    sid = lax.axis_index("s")
    w = cid * _NS + sid
    rows = pl.ds(sid * _RPT, _RPT)

    def stage(grp, slot):
        pltpu.async_copy(src_hbm.at[w, pl.ds(grp * _NBUF2, _NBUF2)],
                         src_v.at[slot], isrc[slot])
        pltpu.async_copy(dst_hbm.at[w, pl.ds(grp * _NBUF2, _NBUF2)],
                         dst_v.at[slot], idst[slot])

    def stage_wait(slot):
        pltpu.make_async_copy(src_hbm.at[w, pl.ds(0, _NBUF2)],
                              src_v.at[slot], isrc[slot]).wait()
        pltpu.make_async_copy(dst_hbm.at[w, pl.ds(0, _NBUF2)],
                              dst_v.at[slot], idst[slot]).wait()

    def g_start(b, slot):
        pltpu.async_copy(tbl_hbm.at[src_v.at[slot, b]], bufs[b], gsems[b])

    def g_wait(b):
        pltpu.make_async_copy(tbl_hbm.at[src_v.at[0, 0]], bufs[b],
                              gsems[b]).wait()

    def s_start(b, slot):
        pltpu.async_copy(bufs[b], acc_s.at[dst_v.at[slot, b]], ssems[b],
                         add=True)

    def s_wait(b):
        pltpu.make_async_copy(bufs[b], acc_s.at[dst_v.at[0, 0]],
                              ssems[b]).wait()

    pltpu.sync_copy(zeros_hbm, acc_s.at[rows])
    stage(0, 0)
    stage(1, 1)
    plsc.subcore_barrier()
    stage_wait(0)
    for b in range(_NBUF2):
        g_start(b, 0)

    def do_round(r, slot, nslot):
        for b in range(_NBUF2):
            g_wait(b)
            s_start(b, slot)

        @pl.when(r + 1 < _NGRP)
        def _():
            stage_wait(nslot)
            for b in range(_NBUF2):
                s_wait(b)
                g_start(b, nslot)

        @pl.when(r + 2 < _NGRP)
        def _():
            stage(r + 2, slot)

    def pair_body(p, carry):
        do_round(2 * p, 0, 1)
        do_round(2 * p + 1, 1, 0)
        return carry

    lax.fori_loop(0, _NGRP // 2, pair_body, 0)
    for b in range(_NBUF2):
        s_wait(b)
    plsc.subcore_barrier()
    pltpu.sync_copy(acc_s.at[rows], s_hbm.at[cid, rows])


@functools.cache
def _make_sc_agg(with_deg):
    if not with_deg:
        scratch = [
            pltpu.VMEM((2, _NBUF2, _CHUNK), jnp.int32),
            pltpu.VMEM((2, _NBUF2, _CHUNK), jnp.int32),
        ]
        scratch += [pltpu.VMEM((_CHUNK, _H), jnp.bfloat16)] * _NBUF2
        scratch.append(pltpu.VMEM_SHARED((_NPAD, _H), jnp.bfloat16))
        scratch += [pltpu.SemaphoreType.DMA] * (2 * _NBUF2 + 4)
        mesh = plsc.VectorSubcoreMesh(core_axis_name="c",
                                      subcore_axis_name="s")
        return pl.kernel(
            _sc_agg_body2,
            out_type=(jax.ShapeDtypeStruct((_NC, _NPAD, _H), jnp.bfloat16),),
            mesh=mesh,
            scratch_types=scratch,
            compiler_params=pltpu.CompilerParams(use_tc_tiling_on_sc=False),
            name="sc_agg",
        )
    scratch = [
        pltpu.VMEM((_NCHUNK, _CHUNK), jnp.int32),
        pltpu.VMEM((_NCHUNK, _CHUNK), jnp.int32),
    ]
    scratch += [pltpu.VMEM((_CHUNK, _H), jnp.bfloat16)] * _NBUF
    out_type = [jax.ShapeDtypeStruct((_NC, _NPAD, _H), jnp.bfloat16)]
    if with_deg:
        scratch.append(pltpu.VMEM((_CHUNK, _CW), jnp.bfloat16))
        out_type.append(jax.ShapeDtypeStruct((_NC, _NPAD, _CW), jnp.bfloat16))
    scratch += [
        pltpu.VMEM_SHARED((_NPAD, _H), jnp.bfloat16),
    ]
    if with_deg:
        scratch.append(pltpu.VMEM_SHARED((_NPAD, _CW), jnp.bfloat16))
    scratch += [pltpu.SemaphoreType.DMA] * (2 * _NBUF)
    if with_deg:
        scratch += [pltpu.SemaphoreType.DMA] * _NBUF
    mesh = plsc.VectorSubcoreMesh(core_axis_name="c", subcore_axis_name="s")
    return pl.kernel(
        functools.partial(_sc_agg_body, with_deg),
        out_type=tuple(out_type),
        mesh=mesh,
        scratch_types=scratch,
        compiler_params=pltpu.CompilerParams(use_tc_tiling_on_sc=False),
        name="sc_agg_deg" if with_deg else "sc_agg",
    )


def _sc_agg_deg(*args):
    return _make_sc_agg(True)(*args)


def _sc_agg(*args):
    return _make_sc_agg(False)(*args)


def _tc_pre_body(x_ref, wl_ref, wr_ref, bl_ref, y_ref, r_ref):
    x = x_ref[...]
    y_ref[...] = _dotT(x, wl_ref[...]).astype(jnp.bfloat16)
    r_ref[...] = (_dotT(x, wr_ref[...]) + bl_ref[...]).astype(jnp.bfloat16)


def _tc_pre(x_pad, wl, wr, bl):
    return pl.pallas_call(
        _tc_pre_body,
        grid=(_GRID,),
        in_specs=[
            pl.BlockSpec((_R, _DIN), lambda i: (i, 0)),
            pl.BlockSpec((_H, _DIN), lambda i: (0, 0)),
            pl.BlockSpec((_H, _DIN), lambda i: (0, 0)),
            pl.BlockSpec((1, _H), lambda i: (0, 0)),
        ],
        out_specs=[
            pl.BlockSpec((_R, _H), lambda i: (i, 0)),
            pl.BlockSpec((_R, _H), lambda i: (i, 0)),
        ],
        out_shape=[
            jax.ShapeDtypeStruct((_NPAD, _H), jnp.bfloat16),
            jax.ShapeDtypeStruct((_NPAD, _H), jnp.bfloat16),
        ],
    )(x_pad, wl, wr, bl)


def _layer_post(s, inv, r, g, b):
    t = ((s[0].astype(jnp.float32) + s[1].astype(jnp.float32))
         * inv.astype(jnp.float32) + r.astype(jnp.float32))
    mu = jnp.mean(t, axis=-1, keepdims=True)
    d = t - mu
    var = jnp.mean(d * d, axis=-1, keepdims=True)
    hn = d * lax.rsqrt(var + 1e-5) * g + b
    return jnp.maximum(hn, 0.0)


def _tc_inv_body(cnt_ref, inv_ref):
    c = (cnt_ref[0, :, :1].astype(jnp.float32)
         + cnt_ref[1, :, :1].astype(jnp.float32))
    inv = 1.0 / jnp.maximum(c, 1.0)
    inv_ref[...] = jnp.broadcast_to(inv, (_R, _H)).astype(jnp.bfloat16)


def _tc_inv(cnt):
    return pl.pallas_call(
        _tc_inv_body,
        grid=(_GRID,),
        in_specs=[pl.BlockSpec((_NC, _R, _CW), lambda i: (0, i, 0))],
        out_specs=pl.BlockSpec((_R, _H), lambda i: (i, 0)),
        out_shape=jax.ShapeDtypeStruct((_NPAD, _H), jnp.bfloat16),
    )(cnt)


def _tc_mid_body(s_ref, cnt_ref, r_ref, g_ref, b_ref, wl_ref, bl_ref,
                 wr_ref, y_ref, r2_ref):
    h = _layer_post(s_ref[...], cnt_ref[...], r_ref[...], g_ref[...],
                    b_ref[...])
    y_ref[...] = _dotT(h, wl_ref[...]).astype(jnp.bfloat16)
    r2_ref[...] = (_dotT(h, wr_ref[...]) + bl_ref[...]).astype(jnp.bfloat16)


def _tc_mid(s, cnt, r, g, b, wl, bl, wr):
    return pl.pallas_call(
        _tc_mid_body,
        grid=(_GRID,),
        in_specs=[
            pl.BlockSpec((_NC, _R, _H), lambda i: (0, i, 0)),
            pl.BlockSpec((_R, _H), lambda i: (i, 0)),
            pl.BlockSpec((_R, _H), lambda i: (i, 0)),
            pl.BlockSpec((1, _H), lambda i: (0, 0)),
            pl.BlockSpec((1, _H), lambda i: (0, 0)),
            pl.BlockSpec((_H, _H), lambda i: (0, 0)),
            pl.BlockSpec((1, _H), lambda i: (0, 0)),
            pl.BlockSpec((_H, _H), lambda i: (0, 0)),
        ],
        out_specs=[
            pl.BlockSpec((_R, _H), lambda i: (i, 0)),
            pl.BlockSpec((_R, _H), lambda i: (i, 0)),
        ],
        out_shape=[
            jax.ShapeDtypeStruct((_NPAD, _H), jnp.bfloat16),
            jax.ShapeDtypeStruct((_NPAD, _H), jnp.bfloat16),
        ],
    )(s, cnt, r, g, b, wl, bl, wr)


def _tc_last_body(s_ref, cnt_ref, r_ref, g_ref, b_ref, wrf_ref, blf_ref,
                  y_ref, r2_ref):
    h = _layer_post(s_ref[...], cnt_ref[...], r_ref[...], g_ref[...],
                    b_ref[...])
    y_ref[...] = h.astype(jnp.bfloat16)
    r2_ref[...] = (_dotT(h, wrf_ref[...]) + blf_ref[...]).astype(jnp.bfloat16)


def _tc_last(s, cnt, r, g, b, wrf, blf):
    return pl.pallas_call(
        _tc_last_body,
        grid=(_GRID,),
        in_specs=[
            pl.BlockSpec((_NC, _R, _H), lambda i: (0, i, 0)),
            pl.BlockSpec((_R, _H), lambda i: (i, 0)),
            pl.BlockSpec((_R, _H), lambda i: (i, 0)),
            pl.BlockSpec((1, _H), lambda i: (0, 0)),
            pl.BlockSpec((1, _H), lambda i: (0, 0)),
            pl.BlockSpec((_DOUT, _H), lambda i: (0, 0)),
            pl.BlockSpec((1, _DOUT), lambda i: (0, 0)),
        ],
        out_specs=[
            pl.BlockSpec((_R, _H), lambda i: (i, 0)),
            pl.BlockSpec((_R, _DOUT), lambda i: (i, 0)),
        ],
        out_shape=[
            jax.ShapeDtypeStruct((_NPAD, _H), jnp.bfloat16),
            jax.ShapeDtypeStruct((_NPAD, _DOUT), jnp.bfloat16),
        ],
    )(s, cnt, r, g, b, wrf, blf)


def _tc_final_body(s_ref, cnt_ref, r_ref, wlf_ref, batch_ref, wo_ref, bo_ref,
                   out_ref, acc, accc):
    step = pl.program_id(0)
    s = s_ref[...].astype(jnp.float32)
    mean = (s[0] + s[1]) * cnt_ref[...].astype(jnp.float32)
    hf = _dotT(mean, wlf_ref[...]) + r_ref[...].astype(jnp.float32)
    bt = batch_ref[...]
    oh = (bt == lax.broadcasted_iota(jnp.int32, (_R, _NG), 1))
    oh = oh.astype(jnp.float32)
    part = lax.dot_general(oh, hf, (((0,), (0,)), ((), ())),
                           preferred_element_type=jnp.float32)
    pc = lax.dot_general(oh, jnp.ones((_R, 1), jnp.float32),
                         (((0,), (0,)), ((), ())),
                         preferred_element_type=jnp.float32)

    @pl.when(step == 0)
    def _():
        acc[...] = jnp.zeros_like(acc)
        accc[...] = jnp.zeros_like(accc)

    acc[...] += part
    accc[...] += pc

    @pl.when(step == pl.num_programs(0) - 1)
    def _():
        gm = acc[...] / jnp.maximum(accc[...], 1.0)
        out_ref[...] = _dotT(gm, wo_ref[...]) + bo_ref[...]


def _tc_final(s, cnt, r, wlf, batch2d, wo, bo):
    return pl.pallas_call(
        _tc_final_body,
        grid=(_GRID,),
        in_specs=[
            pl.BlockSpec((_NC, _R, _H), lambda i: (0, i, 0)),
            pl.BlockSpec((_R, _H), lambda i: (i, 0)),
            pl.BlockSpec((_R, _DOUT), lambda i: (i, 0)),
            pl.BlockSpec((_DOUT, _H), lambda i: (0, 0)),
            pl.BlockSpec((_R, 1), lambda i: (i, 0)),
            pl.BlockSpec((_DOUT, _DOUT), lambda i: (0, 0)),
            pl.BlockSpec((1, _DOUT), lambda i: (0, 0)),
        ],
        out_specs=pl.BlockSpec((_NG, _DOUT), lambda i: (0, 0)),
        out_shape=jax.ShapeDtypeStruct((_NG, _DOUT), jnp.float32),
        scratch_shapes=[
            pltpu.VMEM((_NG, _DOUT), jnp.float32),
            pltpu.VMEM((_NG, 1), jnp.float32),
        ],
    )(s, cnt, r, wlf, batch2d, wo, bo)


def kernel(x, edge_index, batch, params):
    f32 = jnp.float32
    x_pad = jnp.zeros((_NPAD, _DIN), f32).at[:_N].set(x)

    npad_rows = _NPAD - _N
    pad_idx = _N + (jnp.arange(_EPAD - _E, dtype=jnp.int32) % npad_rows)
    src = jnp.concatenate([edge_index[0], pad_idx]).reshape(_NW, _NCHUNK, _CHUNK)
    dst = jnp.concatenate([edge_index[1], pad_idx]).reshape(_NW, _NCHUNK, _CHUNK)

    zeros64 = jnp.zeros((_RPT, _H), jnp.bfloat16)
    zeros1 = jnp.zeros((_RPT, _CW), jnp.bfloat16)
    ones1 = jnp.ones((_CHUNK, _CW), jnp.bfloat16)
    batch2d = jnp.full((_NPAD, 1), _NG, jnp.int32).at[:_N, 0].set(batch)

    def w2(name):
        return params[name]

    def b2(name):
        return params[name].reshape(1, -1)

    y, r = _tc_pre(x_pad, w2("Wl0"), w2("Wr0"), b2("bl0"))
    s, cnt = _sc_agg_deg(y, src, dst, zeros64, ones1, zeros1)
    cnt = _tc_inv(cnt)
    for i in range(1, 5):
        y, r = _tc_mid(s, cnt, r, b2(f"g{i-1}"), b2(f"b{i-1}"),
                       w2(f"Wl{i}"), b2(f"bl{i}"), w2(f"Wr{i}"))
        (s,) = _sc_agg(y, src, dst, zeros64)
    y, r = _tc_last(s, cnt, r, b2("g4"), b2("b4"), w2("Wrf"), b2("blf"))
    (s,) = _sc_agg(y, src, dst, zeros64)
    out = _tc_final(s, cnt, r, w2("Wlf"), batch2d, w2("Wo"), b2("bo"))
    return out
